# baseline (speedup 1.0000x reference)
import functools

import jax
import jax.numpy as jnp
from jax import lax
from jax.experimental import pallas as pl
from jax.experimental.pallas import tpu as pltpu
from jax.experimental.pallas import tpu_sc as plsc

N = 10000
E = 320000
T = 640000
HID = 128
HEADS = 8
DH = HID // HEADS

NC = 2
NS = 16
NW = NC * NS

_SC_MESH = dict(core_axis_name="c", subcore_axis_name="s",
                num_cores=NC, num_subcores=NS)


def _sc_gather(table, idx, sb=400):
    B = idx.shape[0]
    D = table.shape[1]
    per_w = B // NW
    assert per_w * NW == B and per_w % sb == 0 and sb % 8 == 0
    batches = per_w // sb
    if batches % 2:
        sb //= 2
        batches *= 2
    assert batches % 2 == 0 and sb % 8 == 0
    mesh = plsc.VectorSubcoreMesh(**_SC_MESH)

    @functools.partial(
        pl.kernel,
        out_type=jax.ShapeDtypeStruct((B, D), jnp.float32),
        mesh=mesh,
        scratch_types=[
            pltpu.VMEM((per_w,), jnp.int32),
            pltpu.VMEM((sb, D), jnp.float32),
            pltpu.VMEM((sb, D), jnp.float32),
            pltpu.SemaphoreType.DMA,
            pltpu.SemaphoreType.DMA,
        ],
    )
    def gk(table_hbm, idx_hbm, out_hbm, idx_v, rows0, rows1, sem0, sem1):
        wid = lax.axis_index("s") * NC + lax.axis_index("c")
        base = wid * per_w
        pltpu.sync_copy(idx_hbm.at[pl.ds(base, per_w)], idx_v)

        def gat(b, buf, sem):
            return pltpu.make_async_copy(
                table_hbm.at[idx_v.at[pl.ds(b * sb, sb)]], buf, sem)

        pltpu.async_copy(table_hbm.at[idx_v.at[pl.ds(0, sb)]], rows0, sem0)

        def body(j, carry):
            b0 = j * 2
            b1 = b0 + 1
            pltpu.async_copy(
                table_hbm.at[idx_v.at[pl.ds(b1 * sb, sb)]], rows1, sem1)
            gat(b0, rows0, sem0).wait()
            pltpu.sync_copy(rows0, out_hbm.at[pl.ds(base + b0 * sb, sb)])

            @pl.when(b0 + 2 < batches)
            def _():
                pltpu.async_copy(
                    table_hbm.at[idx_v.at[pl.ds((b0 + 2) * sb, sb)]],
                    rows0, sem0)
            gat(b1, rows1, sem1).wait()
            pltpu.sync_copy(rows1, out_hbm.at[pl.ds(base + b1 * sb, sb)])
            return carry

        lax.fori_loop(0, batches // 2, body, 0)

    return gk(table, idx)


def _sc_gather_qkv(q, k, v, idx_kj, idx_ji, sb=80):
    B = idx_kj.shape[0]
    per_w = B // NW
    assert per_w * NW == B and per_w % sb == 0 and sb % 8 == 0
    batches = per_w // sb
    assert batches % 2 == 0
    mesh = plsc.VectorSubcoreMesh(**_SC_MESH)
    rbuf = pltpu.VMEM((sb, HID), jnp.float32)
    dsem = pltpu.SemaphoreType.DMA

    @functools.partial(
        pl.kernel,
        out_type=[jax.ShapeDtypeStruct((B, HID), jnp.float32)] * 3,
        mesh=mesh,
        scratch_types=[
            pltpu.VMEM((per_w,), jnp.int32),
            pltpu.VMEM((per_w,), jnp.int32),
            rbuf, rbuf, rbuf, rbuf, rbuf, rbuf,
            dsem, dsem, dsem, dsem, dsem, dsem,
        ],
    )
    def gk(q_hbm, k_hbm, v_hbm, kj_hbm, ji_hbm, qo_hbm, ko_hbm, vo_hbm,
           kj_v, ji_v, qb0, kb0, vb0, qb1, kb1, vb1,
           qs0, ks0, vs0, qs1, ks1, vs1):
        wid = lax.axis_index("s") * NC + lax.axis_index("c")
        base = wid * per_w
        pltpu.sync_copy(kj_hbm.at[pl.ds(base, per_w)], kj_v)
        pltpu.sync_copy(ji_hbm.at[pl.ds(base, per_w)], ji_v)

        def fire(b, bufs, sems):
            pltpu.async_copy(q_hbm.at[kj_v.at[pl.ds(b * sb, sb)]],
                             bufs[0], sems[0])
            pltpu.async_copy(k_hbm.at[ji_v.at[pl.ds(b * sb, sb)]],
                             bufs[1], sems[1])
            pltpu.async_copy(v_hbm.at[kj_v.at[pl.ds(b * sb, sb)]],
                             bufs[2], sems[2])

        def drain_out(b, bufs, sems):
            pltpu.make_async_copy(
                q_hbm.at[kj_v.at[pl.ds(b * sb, sb)]], bufs[0], sems[0]).wait()
            pltpu.make_async_copy(
                k_hbm.at[ji_v.at[pl.ds(b * sb, sb)]], bufs[1], sems[1]).wait()
            pltpu.make_async_copy(
                v_hbm.at[kj_v.at[pl.ds(b * sb, sb)]], bufs[2], sems[2]).wait()
            sl = pl.ds(base + b * sb, sb)
            pltpu.sync_copy(bufs[0], qo_hbm.at[sl])
            pltpu.sync_copy(bufs[1], ko_hbm.at[sl])
            pltpu.sync_copy(bufs[2], vo_hbm.at[sl])

        bufs0 = (qb0, kb0, vb0)
        sems0 = (qs0, ks0, vs0)
        bufs1 = (qb1, kb1, vb1)
        sems1 = (qs1, ks1, vs1)
        fire(0, bufs0, sems0)

        def body(j, carry):
            b0 = j * 2
            b1 = b0 + 1
            fire(b1, bufs1, sems1)
            drain_out(b0, bufs0, sems0)

            @pl.when(b0 + 2 < batches)
            def _():
                fire(b0 + 2, bufs0, sems0)
            drain_out(b1, bufs1, sems1)
            return carry

        lax.fori_loop(0, batches // 2, body, 0)

    return gk(q, k, v, idx_kj, idx_ji)


def _relu(x):
    return jnp.maximum(x, 0.0)


def _leaky(x):
    return jnp.where(x >= 0, x, 0.2 * x)


def _mm_kernel(x_ref, w_ref, b_ref, o_ref, *, act):
    x = x_ref[...]
    w = w_ref[...]
    y = jax.lax.dot_general(x, w, (((1,), (0,)), ((), ())),
                            preferred_element_type=jnp.float32)
    y = y + b_ref[...]
    if act == "relu":
        y = jnp.maximum(y, 0.0)
    o_ref[...] = y


def _mm(x, w, b=None, act="none", block_rows=512):
    R, K = x.shape
    Kw, Nout = w.shape
    assert K == Kw
    if b is None:
        b = jnp.zeros((Nout,), dtype=jnp.float32)
    pad_r = (-R) % block_rows
    if pad_r:
        x = jnp.pad(x, ((0, pad_r), (0, 0)))
    Rp = R + pad_r
    grid = (Rp // block_rows,)
    out = pl.pallas_call(
        functools.partial(_mm_kernel, act=act),
        grid=grid,
        in_specs=[
            pl.BlockSpec((block_rows, K), lambda i: (i, 0)),
            pl.BlockSpec((K, Nout), lambda i: (0, 0)),
            pl.BlockSpec((Nout,), lambda i: (0,)),
        ],
        out_specs=pl.BlockSpec((block_rows, Nout), lambda i: (i, 0)),
        out_shape=jax.ShapeDtypeStruct((Rp, Nout), jnp.float32),
    )(x, w, b)
    return out[:R] if pad_r else out


def _sc_scatter_rows(values, idx, n_out, sb=80):
    B = values.shape[0]
    per_w = B // NW
    assert per_w * NW == B and per_w % sb == 0 and sb % 8 == 0
    batches = per_w // sb
    n_pad = -(-n_out // (NS * 8)) * (NS * 8)
    per_t = n_pad // NS
    idx3d = idx.reshape(NW, batches, sb)
    zeros = jnp.zeros((per_t, HID), jnp.float32)
    mesh = plsc.VectorSubcoreMesh(**_SC_MESH)

    @functools.partial(
        pl.kernel,
        out_type=jax.ShapeDtypeStruct((NC, n_pad, HID), jnp.float32),
        mesh=mesh,
        scratch_types=[
            pltpu.VMEM_SHARED((n_pad, HID), jnp.float32),
            pltpu.VMEM((sb,), jnp.int32),
            pltpu.VMEM((sb, HID), jnp.float32),
            pltpu.SemaphoreType.DMA,
        ],
    )
    def sk(val_hbm, idx_hbm, z_hbm, out_hbm, slab, idx_v, rows_v, sem):
        cid = lax.axis_index("c")
        sid = lax.axis_index("s")
        wid = sid * NC + cid
        base = wid * per_w

        pltpu.sync_copy(z_hbm, slab.at[pl.ds(sid * per_t, per_t)])
        plsc.subcore_barrier()

        def body(j, carry):
            pltpu.sync_copy(idx_hbm.at[wid].at[j], idx_v)
            pltpu.async_copy(val_hbm.at[pl.ds(base + j * sb, sb)],
                             rows_v, sem).wait()
            pltpu.sync_copy(rows_v, slab.at[idx_v], add=True)
            return carry
        lax.fori_loop(0, batches, body, 0)

        plsc.subcore_barrier()
        pltpu.sync_copy(slab.at[pl.ds(sid * per_t, per_t)],
                        out_hbm.at[cid].at[pl.ds(sid * per_t, per_t)])

    return sk(values, idx3d, zeros)


CH = 8192
NCHUNK = -(-E // CH)
CAP = 768
SBE = 128
CAPB = CAP // SBE
GROWS = 16
PERW_T = T // NW


def _sc_bin(idx_ji):
    stage = 2000
    stages = PERW_T // stage
    mesh = plsc.VectorSubcoreMesh(**_SC_MESH)

    @functools.partial(
        pl.kernel,
        out_type=[
            jax.ShapeDtypeStruct((NW, NCHUNK * CAPB, SBE), jnp.int32),
            jax.ShapeDtypeStruct((NW, NCHUNK * CAPB, SBE), jnp.int32),
            jax.ShapeDtypeStruct((NW, 1, 128), jnp.int32),
        ],
        mesh=mesh,
        compiler_params=pltpu.CompilerParams(needs_layout_passes=False),
        scratch_types=[
            pltpu.VMEM((stage,), jnp.int32),
            pltpu.VMEM((NCHUNK * CAPB, SBE), jnp.int32),
            pltpu.VMEM((NCHUNK * CAPB, SBE), jnp.int32),
            pltpu.VMEM((128,), jnp.int32),
            pltpu.VMEM((1, 128), jnp.int32),
        ],
    )
    def bk(ji_hbm, bt_hbm, bj_hbm, cnt_hbm, jibuf, bt, bj, cnt, cout):
        wid = lax.axis_index("s") * NC + lax.axis_index("c")
        base = wid * PERW_T
        iota = lax.iota(jnp.int32, 16)
        zero16 = jnp.zeros((16,), jnp.int32)

        def zc(i, carry):
            cnt[pl.ds(i * 16, 16)] = zero16
            return carry
        lax.fori_loop(0, 128 // 16, zc, 0)

        def stage_body(s, carry):
            pltpu.sync_copy(ji_hbm.at[pl.ds(base + s * stage, stage)], jibuf)

            lane0 = iota == 0

            def item(i, carry2):
                ji = plsc.load_gather(jibuf, [jnp.full((16,), i, jnp.int32)])
                c = lax.shift_right_logical(ji, 13)
                p = plsc.load_gather(cnt, [c])
                f = c * CAP + jnp.minimum(p, CAP - 1)
                fh = lax.shift_right_logical(f, 7)
                fl = f & (SBE - 1)
                tid = jnp.full((16,), base + s * stage + i, jnp.int32)
                plsc.store_scatter(bt, [fh, fl], tid, mask=lane0)
                plsc.store_scatter(bj, [fh, fl], ji, mask=lane0)
                plsc.addupdate_scatter(cnt, [c], jnp.ones((16,), jnp.int32),
                                       mask=lane0)
                return carry2
            lax.fori_loop(0, stage, item, 0)
            return carry
        lax.fori_loop(0, stages, stage_body, 0)

        def padc(c, carry):
            cvec = jnp.full((16,), c, jnp.int32)
            p = jnp.minimum(jnp.min(plsc.load_gather(cnt, [cvec])), CAP)
            p2 = jnp.minimum(((p + SBE - 1) // SBE) * SBE, CAP)

            def padi(t, carry2):
                q = c * CAP + p + t * 16 + iota
                m = q < c * CAP + p2
                qh = lax.shift_right_logical(q, 7)
                ql = q & (SBE - 1)
                plsc.store_scatter(
                    bt, [qh, ql],
                    wid * 997 + c * 131 + t * 16 + iota, mask=m)
                plsc.store_scatter(
                    bj, [qh, ql],
                    jnp.full((16,), c * CH + CH + (wid & (GROWS - 1)),
                             jnp.int32), mask=m)
                return carry2
            lax.fori_loop(0, (SBE + 15) // 16, padi, 0)
            plsc.store_scatter(cnt, [jnp.full((16,), c, jnp.int32)],
                               jnp.full((16,), p2, jnp.int32),
                               mask=iota == 0)
            return carry
        lax.fori_loop(0, NCHUNK, padc, 0)

        def cw(i, carry):
            cout[0, pl.ds(i * 16, 16)] = cnt[pl.ds(i * 16, 16)]
            return carry
        lax.fori_loop(0, 128 // 16, cw, 0)

        pltpu.sync_copy(bt, bt_hbm.at[wid])
        pltpu.sync_copy(bj, bj_hbm.at[wid])
        pltpu.sync_copy(cout, cnt_hbm.at[wid])

    return bk(idx_ji)


def _sc_agg(v_att, att16, bins_tid, bins_ji, counts):
    stripe = CH // NS
    zeros = jnp.zeros((stripe, HID), jnp.float32)
    zeros16 = jnp.zeros((stripe, 16), jnp.float32)
    mesh = plsc.VectorSubcoreMesh(**_SC_MESH)

    @functools.partial(
        pl.kernel,
        out_type=[jax.ShapeDtypeStruct((E, HID), jnp.float32),
                  jax.ShapeDtypeStruct((E, 16), jnp.float32)],
        mesh=mesh,
        compiler_params=pltpu.CompilerParams(needs_layout_passes=False,
                                             use_tc_tiling_on_sc=False),
        scratch_types=[
            pltpu.VMEM_SHARED((CH + GROWS, HID), jnp.float32),
            pltpu.VMEM_SHARED((CH + GROWS, 16), jnp.float32),
            pltpu.VMEM((NW, 1, 128), jnp.int32),
            pltpu.VMEM((SBE,), jnp.int32),
            pltpu.VMEM((SBE,), jnp.int32),
            pltpu.VMEM((SBE,), jnp.int32),
            pltpu.VMEM((SBE, HID), jnp.float32),
            pltpu.VMEM((SBE, 16), jnp.float32),
            pltpu.SemaphoreType.DMA,
            pltpu.SemaphoreType.DMA,
        ],
    )
    def ek(vatt_hbm, att_hbm, bt_hbm, bj_hbm, cnt_hbm, z_hbm, z16_hbm,
           out_hbm, att_out_hbm,
           slab, aslab, cbuf, tid_v, ji_v, rel_v, rows_v, arows_v,
           sem, asem):
        cid = lax.axis_index("c")
        sid = lax.axis_index("s")
        pltpu.sync_copy(cnt_hbm, cbuf)

        def chunk_body(cc, carry):
            c = cc * NC + cid
            cbase = c * CH

            pltpu.sync_copy(z_hbm, slab.at[pl.ds(sid * stripe, stripe)])
            pltpu.sync_copy(z16_hbm, aslab.at[pl.ds(sid * stripe, stripe)])

            @pl.when(sid == 0)
            def _():
                pltpu.sync_copy(z_hbm.at[pl.ds(0, GROWS)],
                                slab.at[pl.ds(CH, GROWS)])
                pltpu.sync_copy(z16_hbm.at[pl.ds(0, GROWS)],
                                aslab.at[pl.ds(CH, GROWS)])
            plsc.subcore_barrier()

            def drain(wo, carry2):
                w = sid * 2 + wo
                npad = jnp.min(plsc.load_gather(
                    cbuf, [jnp.full((16,), w, jnp.int32),
                           jnp.zeros((16,), jnp.int32),
                           jnp.full((16,), c, jnp.int32)]))
                nb = lax.shift_right_logical(npad, 7)

                def batch(b, carry3):
                    pltpu.sync_copy(bt_hbm.at[w].at[c * CAPB + b], tid_v)
                    pltpu.sync_copy(bj_hbm.at[w].at[c * CAPB + b], ji_v)

                    def torel(i, carry4):
                        rel_v[pl.ds(i * 16, 16)] = (
                            ji_v[pl.ds(i * 16, 16)] - cbase)
                        return carry4
                    lax.fori_loop(0, SBE // 16, torel, 0)
                    cp1 = pltpu.async_copy(vatt_hbm.at[tid_v], rows_v, sem)
                    cp2 = pltpu.async_copy(att_hbm.at[tid_v], arows_v, asem)
                    cp1.wait()
                    cp2.wait()
                    pltpu.sync_copy(rows_v, slab.at[rel_v], add=True)
                    pltpu.sync_copy(arows_v, aslab.at[rel_v], add=True)
                    return carry3
                lax.fori_loop(0, nb, batch, 0)
                return carry2
            lax.fori_loop(0, 2, drain, 0)
            plsc.subcore_barrier()

            rbase = cbase + sid * stripe

            @pl.when(rbase < E)
            def _():
                pltpu.sync_copy(slab.at[pl.ds(sid * stripe, stripe)],
                                out_hbm.at[pl.ds(rbase, stripe)])
                pltpu.sync_copy(aslab.at[pl.ds(sid * stripe, stripe)],
                                att_out_hbm.at[pl.ds(rbase, stripe)])
            return carry

        lax.fori_loop(0, NCHUNK // NC, chunk_body, 0)

    return ek(v_att, att16, bins_tid, bins_ji, counts, zeros, zeros16)


def _pad_rows(x, mult=8):
    pad = (-x.shape[0]) % mult
    return jnp.pad(x, ((0, pad), (0, 0))) if pad else x


def kernel(atom_feature, edge_feature, src, dst, idx_kj, idx_ji, W_i,
           Wv0, Wk0, Wq0, r1w0, r1b0, r2w0, r2b0,
           Wv1, Wk1, Wq1, r1w1, r1b1, r2w1, r2b1,
           W_o, b_o):
    AF = atom_feature.shape[1]

    anode = _mm(atom_feature, W_i[:AF])
    feats = _mm(jnp.pad(edge_feature, ((0, 0), (0, 2))),
                _pad_rows(W_i[AF:]))
    feats = _relu(_sc_gather(anode, src) + feats)
    bins_tid, bins_ji, counts = _sc_bin(idx_ji)

    layers = [(Wv0, Wk0, Wq0, r1w0, r1b0, r2w0, r2b0),
              (Wv1, Wk1, Wq1, r1w1, r1b1, r2w1, r2b1)]
    for (Wv, Wk, Wq, r1w, r1b, r2w, r2b) in layers:
        q = _mm(feats, Wq)
        k = _mm(feats, Wk)
        v = _mm(feats, Wv)
        qg, kg, vg = _sc_gather_qkv(q, k, v, idx_kj, idx_ji)
        att = jnp.sum((qg * kg).reshape(-1, HEADS, DH), axis=-1)
        att = jnp.exp(_leaky(att))
        att16 = jnp.pad(att, ((0, 0), (0, 8)))
        v_att = (vg.reshape(-1, HEADS, DH)
                 * att[:, :, None]).reshape(-1, HID)
        vflat = v
        agg, att_all = _sc_agg(v_att, att16, bins_tid, bins_ji, counts)
        agg = (agg.reshape(-1, HEADS, DH)
               / jnp.maximum(att_all[:, :HEADS], 1e-30)[:, :, None]
               ).reshape(-1, HID)
        h = _mm(agg, r1w, r1b, act="relu")
        feats = vflat + _mm(h, r2w, r2b, act="relu")

    fparts = _sc_scatter_rows(feats, dst, N)
    feats_sum = (fparts[0] + fparts[1])[:N]
    out = _relu(_mm(atom_feature, W_o[:AF]) + _mm(feats_sum, W_o[AF:]) + b_o)
    return out


# speedup vs baseline: 1.3406x; 1.3406x over previous
import functools

import jax
import jax.numpy as jnp
from jax import lax
from jax.experimental import pallas as pl
from jax.experimental.pallas import tpu as pltpu
from jax.experimental.pallas import tpu_sc as plsc

N = 10000
E = 320000
T = 640000
HID = 128
HEADS = 8
DH = HID // HEADS

NC = 2
NS = 16
NW = NC * NS

_SC_MESH = dict(core_axis_name="c", subcore_axis_name="s",
                num_cores=NC, num_subcores=NS)


def _sc_gather(table, idx, sb=400):
    B = idx.shape[0]
    D = table.shape[1]
    per_w = B // NW
    assert per_w * NW == B and per_w % sb == 0 and sb % 8 == 0
    batches = per_w // sb
    if batches % 2:
        sb //= 2
        batches *= 2
    assert batches % 2 == 0 and sb % 8 == 0
    mesh = plsc.VectorSubcoreMesh(**_SC_MESH)

    @functools.partial(
        pl.kernel,
        out_type=jax.ShapeDtypeStruct((B, D), jnp.float32),
        mesh=mesh,
        scratch_types=[
            pltpu.VMEM((per_w,), jnp.int32),
            pltpu.VMEM((sb, D), jnp.float32),
            pltpu.VMEM((sb, D), jnp.float32),
            pltpu.SemaphoreType.DMA,
            pltpu.SemaphoreType.DMA,
        ],
    )
    def gk(table_hbm, idx_hbm, out_hbm, idx_v, rows0, rows1, sem0, sem1):
        wid = lax.axis_index("s") * NC + lax.axis_index("c")
        base = wid * per_w
        pltpu.sync_copy(idx_hbm.at[pl.ds(base, per_w)], idx_v)

        def gat(b, buf, sem):
            return pltpu.make_async_copy(
                table_hbm.at[idx_v.at[pl.ds(b * sb, sb)]], buf, sem)

        pltpu.async_copy(table_hbm.at[idx_v.at[pl.ds(0, sb)]], rows0, sem0)

        def body(j, carry):
            b0 = j * 2
            b1 = b0 + 1
            pltpu.async_copy(
                table_hbm.at[idx_v.at[pl.ds(b1 * sb, sb)]], rows1, sem1)
            gat(b0, rows0, sem0).wait()
            pltpu.sync_copy(rows0, out_hbm.at[pl.ds(base + b0 * sb, sb)])

            @pl.when(b0 + 2 < batches)
            def _():
                pltpu.async_copy(
                    table_hbm.at[idx_v.at[pl.ds((b0 + 2) * sb, sb)]],
                    rows0, sem0)
            gat(b1, rows1, sem1).wait()
            pltpu.sync_copy(rows1, out_hbm.at[pl.ds(base + b1 * sb, sb)])
            return carry

        lax.fori_loop(0, batches // 2, body, 0)

    return gk(table, idx)


def _relu(x):
    return jnp.maximum(x, 0.0)


def _leaky(x):
    return jnp.where(x >= 0, x, 0.2 * x)


def _mm_kernel(x_ref, w_ref, b_ref, o_ref, *, act):
    x = x_ref[...]
    w = w_ref[...]
    y = jax.lax.dot_general(x, w, (((1,), (0,)), ((), ())),
                            preferred_element_type=jnp.float32)
    y = y + b_ref[...]
    if act == "relu":
        y = jnp.maximum(y, 0.0)
    o_ref[...] = y


def _mm(x, w, b=None, act="none", block_rows=512):
    R, K = x.shape
    Kw, Nout = w.shape
    assert K == Kw
    if b is None:
        b = jnp.zeros((Nout,), dtype=jnp.float32)
    pad_r = (-R) % block_rows
    if pad_r:
        x = jnp.pad(x, ((0, pad_r), (0, 0)))
    Rp = R + pad_r
    grid = (Rp // block_rows,)
    out = pl.pallas_call(
        functools.partial(_mm_kernel, act=act),
        grid=grid,
        in_specs=[
            pl.BlockSpec((block_rows, K), lambda i: (i, 0)),
            pl.BlockSpec((K, Nout), lambda i: (0, 0)),
            pl.BlockSpec((Nout,), lambda i: (0,)),
        ],
        out_specs=pl.BlockSpec((block_rows, Nout), lambda i: (i, 0)),
        out_shape=jax.ShapeDtypeStruct((Rp, Nout), jnp.float32),
    )(x, w, b)
    return out[:R] if pad_r else out


def _mm3_kernel(x_ref, w_ref, o1_ref, o2_ref, o3_ref):
    y = jax.lax.dot_general(x_ref[...], w_ref[...], (((1,), (0,)), ((), ())),
                            preferred_element_type=jnp.float32)
    o1_ref[...] = y[:, :HID]
    o2_ref[...] = y[:, HID:2 * HID]
    o3_ref[...] = y[:, 2 * HID:]


def _mm3(x, w1, w2, w3, block_rows=1000):
    R, K = x.shape
    w = jnp.concatenate([w1, w2, w3], axis=1)
    assert R % block_rows == 0
    grid = (R // block_rows,)
    return pl.pallas_call(
        _mm3_kernel,
        grid=grid,
        in_specs=[
            pl.BlockSpec((block_rows, K), lambda i: (i, 0)),
            pl.BlockSpec((K, 3 * HID), lambda i: (0, 0)),
        ],
        out_specs=[pl.BlockSpec((block_rows, HID), lambda i: (i, 0))] * 3,
        out_shape=[jax.ShapeDtypeStruct((R, HID), jnp.float32)] * 3,
    )(x, w)


def _res_kernel(x_ref, v_ref, w1_ref, b1_ref, w2_ref, b2_ref, o_ref):
    h = jax.lax.dot_general(x_ref[...], w1_ref[...], (((1,), (0,)), ((), ())),
                            preferred_element_type=jnp.float32)
    h = jnp.maximum(h + b1_ref[...], 0.0)
    y = jax.lax.dot_general(h, w2_ref[...], (((1,), (0,)), ((), ())),
                            preferred_element_type=jnp.float32)
    o_ref[...] = v_ref[...] + jnp.maximum(y + b2_ref[...], 0.0)


def _residual(x, vflat, w1, b1, w2, b2, block_rows=1000):
    R, K = x.shape
    assert R % block_rows == 0
    grid = (R // block_rows,)
    return pl.pallas_call(
        _res_kernel,
        grid=grid,
        in_specs=[
            pl.BlockSpec((block_rows, K), lambda i: (i, 0)),
            pl.BlockSpec((block_rows, HID), lambda i: (i, 0)),
            pl.BlockSpec((K, HID), lambda i: (0, 0)),
            pl.BlockSpec((HID,), lambda i: (0,)),
            pl.BlockSpec((HID, HID), lambda i: (0, 0)),
            pl.BlockSpec((HID,), lambda i: (0,)),
        ],
        out_specs=pl.BlockSpec((block_rows, HID), lambda i: (i, 0)),
        out_shape=jax.ShapeDtypeStruct((R, HID), jnp.float32),
    )(x, vflat, w1, b1, w2, b2)


def _out_kernel(a_ref, f0_ref, f1_ref, w1_ref, w2_ref, b_ref, o_ref):
    y = jax.lax.dot_general(a_ref[...], w1_ref[...], (((1,), (0,)), ((), ())),
                            preferred_element_type=jnp.float32)
    f = f0_ref[...] + f1_ref[...]
    y = y + jax.lax.dot_general(f, w2_ref[...], (((1,), (0,)), ((), ())),
                                preferred_element_type=jnp.float32)
    o_ref[...] = jnp.maximum(y + b_ref[...], 0.0)


def _out_proj(atom, f0, f1, w1, w2, b, block_rows=400):
    R, K = atom.shape
    assert R % block_rows == 0
    grid = (R // block_rows,)
    return pl.pallas_call(
        _out_kernel,
        grid=grid,
        in_specs=[
            pl.BlockSpec((block_rows, K), lambda i: (i, 0)),
            pl.BlockSpec((block_rows, HID), lambda i: (i, 0)),
            pl.BlockSpec((block_rows, HID), lambda i: (i, 0)),
            pl.BlockSpec((K, HID), lambda i: (0, 0)),
            pl.BlockSpec((HID, HID), lambda i: (0, 0)),
            pl.BlockSpec((HID,), lambda i: (0,)),
        ],
        out_specs=pl.BlockSpec((block_rows, HID), lambda i: (i, 0)),
        out_shape=jax.ShapeDtypeStruct((R, HID), jnp.float32),
    )(atom, f0, f1, w1, w2, b)


def _sc_scatter_rows(values, idx, n_out, sb=80):
    B = values.shape[0]
    per_w = B // NW
    assert per_w * NW == B and per_w % sb == 0 and sb % 8 == 0
    batches = per_w // sb
    n_pad = -(-n_out // (NS * 8)) * (NS * 8)
    per_t = n_pad // NS
    idx3d = idx.reshape(NW, batches, sb)
    zeros = jnp.zeros((per_t, HID), jnp.float32)
    mesh = plsc.VectorSubcoreMesh(**_SC_MESH)

    @functools.partial(
        pl.kernel,
        out_type=jax.ShapeDtypeStruct((NC, n_pad, HID), jnp.float32),
        mesh=mesh,
        scratch_types=[
            pltpu.VMEM_SHARED((n_pad, HID), jnp.float32),
            pltpu.VMEM((sb,), jnp.int32),
            pltpu.VMEM((sb, HID), jnp.float32),
            pltpu.SemaphoreType.DMA,
        ],
    )
    def sk(val_hbm, idx_hbm, z_hbm, out_hbm, slab, idx_v, rows_v, sem):
        cid = lax.axis_index("c")
        sid = lax.axis_index("s")
        wid = sid * NC + cid
        base = wid * per_w

        pltpu.sync_copy(z_hbm, slab.at[pl.ds(sid * per_t, per_t)])
        plsc.subcore_barrier()

        def body(j, carry):
            pltpu.sync_copy(idx_hbm.at[wid].at[j], idx_v)
            pltpu.async_copy(val_hbm.at[pl.ds(base + j * sb, sb)],
                             rows_v, sem).wait()
            pltpu.sync_copy(rows_v, slab.at[idx_v], add=True)
            return carry
        lax.fori_loop(0, batches, body, 0)

        plsc.subcore_barrier()
        pltpu.sync_copy(slab.at[pl.ds(sid * per_t, per_t)],
                        out_hbm.at[cid].at[pl.ds(sid * per_t, per_t)])

    return sk(values, idx3d, zeros)


CH = 8192
NCHUNK = -(-E // CH)
CAP = 768
SBE = 128
CAPB = CAP // SBE
GROWS = 16
PERW_T = T // NW


def _sc_bin(idx_ji):
    stage = 2000
    stages = PERW_T // stage
    mesh = plsc.VectorSubcoreMesh(**_SC_MESH)

    @functools.partial(
        pl.kernel,
        out_type=[
            jax.ShapeDtypeStruct((NW, NCHUNK * CAPB, SBE), jnp.int32),
            jax.ShapeDtypeStruct((NW, NCHUNK * CAPB, SBE), jnp.int32),
            jax.ShapeDtypeStruct((NW, 1, 128), jnp.int32),
        ],
        mesh=mesh,
        compiler_params=pltpu.CompilerParams(needs_layout_passes=False),
        scratch_types=[
            pltpu.VMEM((stage,), jnp.int32),
            pltpu.VMEM((NCHUNK * CAPB, SBE), jnp.int32),
            pltpu.VMEM((NCHUNK * CAPB, SBE), jnp.int32),
            pltpu.VMEM((128,), jnp.int32),
            pltpu.VMEM((1, 128), jnp.int32),
        ],
    )
    def bk(ji_hbm, bt_hbm, bj_hbm, cnt_hbm, jibuf, bt, bj, cnt, cout):
        wid = lax.axis_index("s") * NC + lax.axis_index("c")
        base = wid * PERW_T
        iota = lax.iota(jnp.int32, 16)
        zero16 = jnp.zeros((16,), jnp.int32)

        def zc(i, carry):
            cnt[pl.ds(i * 16, 16)] = zero16
            return carry
        lax.fori_loop(0, 128 // 16, zc, 0)

        def stage_body(s, carry):
            pltpu.sync_copy(ji_hbm.at[pl.ds(base + s * stage, stage)], jibuf)

            lane0 = iota == 0

            def item(i, carry2):
                ji = plsc.load_gather(jibuf, [jnp.full((16,), i, jnp.int32)])
                c = lax.shift_right_logical(ji, 13)
                p = plsc.load_gather(cnt, [c])
                f = c * CAP + jnp.minimum(p, CAP - 1)
                fh = lax.shift_right_logical(f, 7)
                fl = f & (SBE - 1)
                tid = jnp.full((16,), base + s * stage + i, jnp.int32)
                plsc.store_scatter(bt, [fh, fl], tid, mask=lane0)
                plsc.store_scatter(bj, [fh, fl], ji, mask=lane0)
                plsc.addupdate_scatter(cnt, [c], jnp.ones((16,), jnp.int32),
                                       mask=lane0)
                return carry2
            lax.fori_loop(0, stage, item, 0)
            return carry
        lax.fori_loop(0, stages, stage_body, 0)

        def padc(c, carry):
            cvec = jnp.full((16,), c, jnp.int32)
            p = jnp.minimum(jnp.min(plsc.load_gather(cnt, [cvec])), CAP)
            p2 = jnp.minimum(((p + SBE - 1) // SBE) * SBE, CAP)

            def padi(t, carry2):
                q = c * CAP + p + t * 16 + iota
                m = q < c * CAP + p2
                qh = lax.shift_right_logical(q, 7)
                ql = q & (SBE - 1)
                plsc.store_scatter(
                    bt, [qh, ql],
                    wid * 997 + c * 131 + t * 16 + iota, mask=m)
                plsc.store_scatter(
                    bj, [qh, ql],
                    jnp.full((16,), c * CH + CH + (wid & (GROWS - 1)),
                             jnp.int32), mask=m)
                return carry2
            lax.fori_loop(0, (SBE + 15) // 16, padi, 0)
            plsc.store_scatter(cnt, [jnp.full((16,), c, jnp.int32)],
                               jnp.full((16,), p2, jnp.int32),
                               mask=iota == 0)
            return carry
        lax.fori_loop(0, NCHUNK, padc, 0)

        def cw(i, carry):
            cout[0, pl.ds(i * 16, 16)] = cnt[pl.ds(i * 16, 16)]
            return carry
        lax.fori_loop(0, 128 // 16, cw, 0)

        pltpu.sync_copy(bt, bt_hbm.at[wid])
        pltpu.sync_copy(bj, bj_hbm.at[wid])
        pltpu.sync_copy(cout, cnt_hbm.at[wid])

    return bk(idx_ji)


def _sc_agg(v_att, att8, bins_tid, bins_ji, counts):
    stripe = CH // NS
    zeros = jnp.zeros((stripe, HID), jnp.float32)
    zeros8 = jnp.zeros((stripe, HEADS), jnp.float32)
    mesh = plsc.VectorSubcoreMesh(**_SC_MESH)

    @functools.partial(
        pl.kernel,
        out_type=[jax.ShapeDtypeStruct((E, HID), jnp.float32),
                  jax.ShapeDtypeStruct((E, HEADS), jnp.float32)],
        mesh=mesh,
        compiler_params=pltpu.CompilerParams(needs_layout_passes=False,
                                             use_tc_tiling_on_sc=False),
        scratch_types=[
            pltpu.VMEM_SHARED((CH + GROWS, HID), jnp.float32),
            pltpu.VMEM_SHARED((CH + GROWS, HEADS), jnp.float32),
            pltpu.VMEM((NW, 1, 128), jnp.int32),
            pltpu.VMEM((SBE,), jnp.int32),
            pltpu.VMEM((SBE,), jnp.int32),
            pltpu.VMEM((SBE,), jnp.int32),
            pltpu.VMEM((SBE, HID), jnp.float32),
            pltpu.VMEM((SBE, HEADS), jnp.float32),
            pltpu.SemaphoreType.DMA,
            pltpu.SemaphoreType.DMA,
        ],
    )
    def ek(vatt_hbm, att_hbm, bt_hbm, bj_hbm, cnt_hbm, z_hbm, z16_hbm,
           out_hbm, att_out_hbm,
           slab, aslab, cbuf, tid_v, ji_v, rel_v, rows_v, arows_v,
           sem, asem):
        cid = lax.axis_index("c")
        sid = lax.axis_index("s")
        pltpu.sync_copy(cnt_hbm, cbuf)

        def chunk_body(cc, carry):
            c = cc * NC + cid
            cbase = c * CH

            pltpu.sync_copy(z_hbm, slab.at[pl.ds(sid * stripe, stripe)])
            pltpu.sync_copy(z16_hbm, aslab.at[pl.ds(sid * stripe, stripe)])

            @pl.when(sid == 0)
            def _():
                pltpu.sync_copy(z_hbm.at[pl.ds(0, GROWS)],
                                slab.at[pl.ds(CH, GROWS)])
                pltpu.sync_copy(z16_hbm.at[pl.ds(0, GROWS)],
                                aslab.at[pl.ds(CH, GROWS)])
            plsc.subcore_barrier()

            def drain(wo, carry2):
                w = sid * 2 + wo
                npad = jnp.min(plsc.load_gather(
                    cbuf, [jnp.full((16,), w, jnp.int32),
                           jnp.zeros((16,), jnp.int32),
                           jnp.full((16,), c, jnp.int32)]))
                nb = lax.shift_right_logical(npad, 7)

                def batch(b, carry3):
                    pltpu.sync_copy(bt_hbm.at[w].at[c * CAPB + b], tid_v)
                    pltpu.sync_copy(bj_hbm.at[w].at[c * CAPB + b], ji_v)

                    def torel(i, carry4):
                        rel_v[pl.ds(i * 16, 16)] = (
                            ji_v[pl.ds(i * 16, 16)] - cbase)
                        return carry4
                    lax.fori_loop(0, SBE // 16, torel, 0)
                    cp1 = pltpu.async_copy(vatt_hbm.at[tid_v], rows_v, sem)
                    cp2 = pltpu.async_copy(att_hbm.at[tid_v], arows_v, asem)
                    cp1.wait()
                    cp2.wait()
                    pltpu.sync_copy(rows_v, slab.at[rel_v], add=True)
                    pltpu.sync_copy(arows_v, aslab.at[rel_v], add=True)
                    return carry3
                lax.fori_loop(0, nb, batch, 0)
                return carry2
            lax.fori_loop(0, 2, drain, 0)
            plsc.subcore_barrier()

            rbase = cbase + sid * stripe

            @pl.when(rbase < E)
            def _():
                pltpu.sync_copy(slab.at[pl.ds(sid * stripe, stripe)],
                                out_hbm.at[pl.ds(rbase, stripe)])
                pltpu.sync_copy(aslab.at[pl.ds(sid * stripe, stripe)],
                                att_out_hbm.at[pl.ds(rbase, stripe)])
            return carry

        lax.fori_loop(0, NCHUNK // NC, chunk_body, 0)

    return ek(v_att, att8, bins_tid, bins_ji, counts, zeros, zeros8)


def _pad_rows(x, mult=8):
    pad = (-x.shape[0]) % mult
    return jnp.pad(x, ((0, pad), (0, 0))) if pad else x


def kernel(atom_feature, edge_feature, src, dst, idx_kj, idx_ji, W_i,
           Wv0, Wk0, Wq0, r1w0, r1b0, r2w0, r2b0,
           Wv1, Wk1, Wq1, r1w1, r1b1, r2w1, r2b1,
           W_o, b_o):
    AF = atom_feature.shape[1]

    anode = _mm(atom_feature, W_i[:AF])
    feats = _mm(jnp.pad(edge_feature, ((0, 0), (0, 2))),
                _pad_rows(W_i[AF:]))
    feats = _relu(_sc_gather(anode, src) + feats)
    bins_tid, bins_ji, counts = _sc_bin(idx_ji)

    layers = [(Wv0, Wk0, Wq0, r1w0, r1b0, r2w0, r2b0),
              (Wv1, Wk1, Wq1, r1w1, r1b1, r2w1, r2b1)]
    for (Wv, Wk, Wq, r1w, r1b, r2w, r2b) in layers:
        q, k, v = _mm3(feats, Wq, Wk, Wv)
        qg = _sc_gather(q, idx_kj)
        kg = _sc_gather(k, idx_ji)
        att = jnp.sum((qg * kg).reshape(-1, HEADS, DH), axis=-1)
        att = jnp.exp(_leaky(att))
        vg = _sc_gather(v, idx_kj)
        v_att = (vg.reshape(-1, HEADS, DH)
                 * att[:, :, None]).reshape(-1, HID)
        agg, att_all = _sc_agg(v_att, att, bins_tid, bins_ji, counts)
        agg = (agg.reshape(-1, HEADS, DH)
               / jnp.maximum(att_all, 1e-30)[:, :, None]
               ).reshape(-1, HID)
        feats = _residual(agg, v, r1w, r1b, r2w, r2b)

    fparts = _sc_scatter_rows(feats, dst, N)
    out = _out_proj(jnp.pad(atom_feature, ((0, 0), (0, 3))),
                    fparts[0][:N], fparts[1][:N],
                    jnp.pad(W_o[:AF], ((0, 3), (0, 0))), W_o[AF:], b_o)
    return out


# speedup vs baseline: 1.7994x; 1.3422x over previous
import functools

import jax
import jax.numpy as jnp
from jax import lax
from jax.experimental import pallas as pl
from jax.experimental.pallas import tpu as pltpu
from jax.experimental.pallas import tpu_sc as plsc

N = 10000
E = 320000
T = 640000
HID = 128
HEADS = 8
DH = HID // HEADS

NC = 2
NS = 16
NW = NC * NS

_SC_MESH = dict(core_axis_name="c", subcore_axis_name="s",
                num_cores=NC, num_subcores=NS)


def _sc_gather(table, idx, sb=400):
    B = idx.shape[0]
    D = table.shape[1]
    per_w = B // NW
    assert per_w * NW == B and per_w % sb == 0 and sb % 8 == 0
    batches = per_w // sb
    if batches % 2:
        sb //= 2
        batches *= 2
    assert batches % 2 == 0 and sb % 8 == 0
    mesh = plsc.VectorSubcoreMesh(**_SC_MESH)

    @functools.partial(
        pl.kernel,
        out_type=jax.ShapeDtypeStruct((B, D), jnp.float32),
        mesh=mesh,
        scratch_types=[
            pltpu.VMEM((per_w,), jnp.int32),
            pltpu.VMEM((sb, D), jnp.float32),
            pltpu.VMEM((sb, D), jnp.float32),
            pltpu.SemaphoreType.DMA,
            pltpu.SemaphoreType.DMA,
        ],
    )
    def gk(table_hbm, idx_hbm, out_hbm, idx_v, rows0, rows1, sem0, sem1):
        wid = lax.axis_index("s") * NC + lax.axis_index("c")
        base = wid * per_w
        pltpu.sync_copy(idx_hbm.at[pl.ds(base, per_w)], idx_v)

        def gat(b, buf, sem):
            return pltpu.make_async_copy(
                table_hbm.at[idx_v.at[pl.ds(b * sb, sb)]], buf, sem)

        pltpu.async_copy(table_hbm.at[idx_v.at[pl.ds(0, sb)]], rows0, sem0)

        def body(j, carry):
            b0 = j * 2
            b1 = b0 + 1
            pltpu.async_copy(
                table_hbm.at[idx_v.at[pl.ds(b1 * sb, sb)]], rows1, sem1)
            gat(b0, rows0, sem0).wait()
            pltpu.sync_copy(rows0, out_hbm.at[pl.ds(base + b0 * sb, sb)])

            @pl.when(b0 + 2 < batches)
            def _():
                pltpu.async_copy(
                    table_hbm.at[idx_v.at[pl.ds((b0 + 2) * sb, sb)]],
                    rows0, sem0)
            gat(b1, rows1, sem1).wait()
            pltpu.sync_copy(rows1, out_hbm.at[pl.ds(base + b1 * sb, sb)])
            return carry

        lax.fori_loop(0, batches // 2, body, 0)

    return gk(table, idx)


def _relu(x):
    return jnp.maximum(x, 0.0)


def _leaky(x):
    return jnp.where(x >= 0, x, 0.2 * x)


def _mm_kernel(x_ref, w_ref, b_ref, o_ref, *, act):
    x = x_ref[...]
    w = w_ref[...]
    y = jax.lax.dot_general(x, w, (((1,), (0,)), ((), ())),
                            preferred_element_type=jnp.float32)
    y = y + b_ref[...]
    if act == "relu":
        y = jnp.maximum(y, 0.0)
    o_ref[...] = y


def _mm(x, w, b=None, act="none", block_rows=512):
    R, K = x.shape
    Kw, Nout = w.shape
    assert K == Kw
    if b is None:
        b = jnp.zeros((Nout,), dtype=jnp.float32)
    pad_r = (-R) % block_rows
    if pad_r:
        x = jnp.pad(x, ((0, pad_r), (0, 0)))
    Rp = R + pad_r
    grid = (Rp // block_rows,)
    out = pl.pallas_call(
        functools.partial(_mm_kernel, act=act),
        grid=grid,
        in_specs=[
            pl.BlockSpec((block_rows, K), lambda i: (i, 0)),
            pl.BlockSpec((K, Nout), lambda i: (0, 0)),
            pl.BlockSpec((Nout,), lambda i: (0,)),
        ],
        out_specs=pl.BlockSpec((block_rows, Nout), lambda i: (i, 0)),
        out_shape=jax.ShapeDtypeStruct((Rp, Nout), jnp.float32),
    )(x, w, b)
    return out[:R] if pad_r else out


def _mm3_kernel(x_ref, w_ref, o1_ref, o2_ref, o3_ref):
    y = jax.lax.dot_general(x_ref[...], w_ref[...], (((1,), (0,)), ((), ())),
                            preferred_element_type=jnp.float32)
    o1_ref[...] = y[:, :HID]
    o2_ref[...] = y[:, HID:2 * HID]
    o3_ref[...] = y[:, 2 * HID:]


def _mm3(x, w1, w2, w3, block_rows=1000):
    R, K = x.shape
    w = jnp.concatenate([w1, w2, w3], axis=1)
    assert R % block_rows == 0
    grid = (R // block_rows,)
    return pl.pallas_call(
        _mm3_kernel,
        grid=grid,
        in_specs=[
            pl.BlockSpec((block_rows, K), lambda i: (i, 0)),
            pl.BlockSpec((K, 3 * HID), lambda i: (0, 0)),
        ],
        out_specs=[pl.BlockSpec((block_rows, HID), lambda i: (i, 0))] * 3,
        out_shape=[jax.ShapeDtypeStruct((R, HID), jnp.float32)] * 3,
    )(x, w)


def _res_kernel(x_ref, d_ref, v_ref, st_ref, w1_ref, b1_ref, w2_ref, b2_ref,
                o_ref):
    den = jax.lax.dot_general(d_ref[...], st_ref[...],
                              (((1,), (0,)), ((), ())),
                              preferred_element_type=jnp.float32)
    x = x_ref[...] / jnp.maximum(den, 1e-30)
    h = jax.lax.dot_general(x, w1_ref[...], (((1,), (0,)), ((), ())),
                            preferred_element_type=jnp.float32)
    h = jnp.maximum(h + b1_ref[...], 0.0)
    y = jax.lax.dot_general(h, w2_ref[...], (((1,), (0,)), ((), ())),
                            preferred_element_type=jnp.float32)
    o_ref[...] = v_ref[...] + jnp.maximum(y + b2_ref[...], 0.0)


def _residual(x, att_all, vflat, w1, b1, w2, b2, block_rows=1000):
    R, K = x.shape
    assert R % block_rows == 0
    sel = (jnp.arange(HID)[:, None] // DH
           == jnp.arange(HEADS)[None, :]).astype(jnp.float32)
    grid = (R // block_rows,)
    return pl.pallas_call(
        _res_kernel,
        grid=grid,
        in_specs=[
            pl.BlockSpec((block_rows, K), lambda i: (i, 0)),
            pl.BlockSpec((block_rows, HEADS), lambda i: (i, 0)),
            pl.BlockSpec((block_rows, HID), lambda i: (i, 0)),
            pl.BlockSpec((HEADS, HID), lambda i: (0, 0)),
            pl.BlockSpec((K, HID), lambda i: (0, 0)),
            pl.BlockSpec((HID,), lambda i: (0,)),
            pl.BlockSpec((HID, HID), lambda i: (0, 0)),
            pl.BlockSpec((HID,), lambda i: (0,)),
        ],
        out_specs=pl.BlockSpec((block_rows, HID), lambda i: (i, 0)),
        out_shape=jax.ShapeDtypeStruct((R, HID), jnp.float32),
    )(x, att_all, vflat, sel.T, w1, b1, w2, b2)


def _att_kernel(q_ref, k_ref, v_ref, s_ref, st_ref, a_ref, va_ref):
    qk = q_ref[...] * k_ref[...]
    a = jax.lax.dot_general(qk, s_ref[...], (((1,), (0,)), ((), ())),
                            preferred_element_type=jnp.float32)
    a = jnp.exp(jnp.where(a >= 0, a, 0.2 * a))
    a_ref[...] = a
    ab = jax.lax.dot_general(a, st_ref[...], (((1,), (0,)), ((), ())),
                             preferred_element_type=jnp.float32)
    va_ref[...] = v_ref[...] * ab


def _att_vatt(qg, kg, vg, block_rows=1000):
    R = qg.shape[0]
    assert R % block_rows == 0
    sel = (jnp.arange(HID)[:, None] // DH
           == jnp.arange(HEADS)[None, :]).astype(jnp.float32)
    grid = (R // block_rows,)
    return pl.pallas_call(
        _att_kernel,
        grid=grid,
        in_specs=[
            pl.BlockSpec((block_rows, HID), lambda i: (i, 0)),
            pl.BlockSpec((block_rows, HID), lambda i: (i, 0)),
            pl.BlockSpec((block_rows, HID), lambda i: (i, 0)),
            pl.BlockSpec((HID, HEADS), lambda i: (0, 0)),
            pl.BlockSpec((HEADS, HID), lambda i: (0, 0)),
        ],
        out_specs=[pl.BlockSpec((block_rows, HEADS), lambda i: (i, 0)),
                   pl.BlockSpec((block_rows, HID), lambda i: (i, 0))],
        out_shape=[jax.ShapeDtypeStruct((R, HEADS), jnp.float32),
                   jax.ShapeDtypeStruct((R, HID), jnp.float32)],
    )(qg, kg, vg, sel, sel.T)


def _out_kernel(a_ref, f0_ref, f1_ref, w1_ref, w2_ref, b_ref, o_ref):
    y = jax.lax.dot_general(a_ref[...], w1_ref[...], (((1,), (0,)), ((), ())),
                            preferred_element_type=jnp.float32)
    f = f0_ref[...] + f1_ref[...]
    y = y + jax.lax.dot_general(f, w2_ref[...], (((1,), (0,)), ((), ())),
                                preferred_element_type=jnp.float32)
    o_ref[...] = jnp.maximum(y + b_ref[...], 0.0)


def _out_proj(atom, f0, f1, w1, w2, b, block_rows=400):
    R, K = atom.shape
    assert R % block_rows == 0
    grid = (R // block_rows,)
    return pl.pallas_call(
        _out_kernel,
        grid=grid,
        in_specs=[
            pl.BlockSpec((block_rows, K), lambda i: (i, 0)),
            pl.BlockSpec((block_rows, HID), lambda i: (i, 0)),
            pl.BlockSpec((block_rows, HID), lambda i: (i, 0)),
            pl.BlockSpec((K, HID), lambda i: (0, 0)),
            pl.BlockSpec((HID, HID), lambda i: (0, 0)),
            pl.BlockSpec((HID,), lambda i: (0,)),
        ],
        out_specs=pl.BlockSpec((block_rows, HID), lambda i: (i, 0)),
        out_shape=jax.ShapeDtypeStruct((R, HID), jnp.float32),
    )(atom, f0, f1, w1, w2, b)


def _sc_scatter_rows(values, idx, n_out, sb=80):
    B = values.shape[0]
    per_w = B // NW
    assert per_w * NW == B and per_w % sb == 0 and sb % 8 == 0
    batches = per_w // sb
    n_pad = -(-n_out // (NS * 8)) * (NS * 8)
    per_t = n_pad // NS
    idx3d = idx.reshape(NW, batches, sb)
    zeros = jnp.zeros((per_t, HID), jnp.float32)
    mesh = plsc.VectorSubcoreMesh(**_SC_MESH)

    @functools.partial(
        pl.kernel,
        out_type=jax.ShapeDtypeStruct((NC, n_pad, HID), jnp.float32),
        mesh=mesh,
        scratch_types=[
            pltpu.VMEM_SHARED((n_pad, HID), jnp.float32),
            pltpu.VMEM((sb,), jnp.int32),
            pltpu.VMEM((sb, HID), jnp.float32),
            pltpu.SemaphoreType.DMA,
        ],
    )
    def sk(val_hbm, idx_hbm, z_hbm, out_hbm, slab, idx_v, rows_v, sem):
        cid = lax.axis_index("c")
        sid = lax.axis_index("s")
        wid = sid * NC + cid
        base = wid * per_w

        pltpu.sync_copy(z_hbm, slab.at[pl.ds(sid * per_t, per_t)])
        plsc.subcore_barrier()

        def body(j, carry):
            pltpu.sync_copy(idx_hbm.at[wid].at[j], idx_v)
            pltpu.async_copy(val_hbm.at[pl.ds(base + j * sb, sb)],
                             rows_v, sem).wait()
            pltpu.sync_copy(rows_v, slab.at[idx_v], add=True)
            return carry
        lax.fori_loop(0, batches, body, 0)

        plsc.subcore_barrier()
        pltpu.sync_copy(slab.at[pl.ds(sid * per_t, per_t)],
                        out_hbm.at[cid].at[pl.ds(sid * per_t, per_t)])

    return sk(values, idx3d, zeros)


CH = 8192
NCHUNK = -(-E // CH)
CAP = 768
SBE = 128
CAPB = CAP // SBE
GROWS = 16
PERW_T = T // NW


def _sc_bin(idx_ji):
    stage = 2000
    stages = PERW_T // stage
    mesh = plsc.VectorSubcoreMesh(**_SC_MESH)

    @functools.partial(
        pl.kernel,
        out_type=[
            jax.ShapeDtypeStruct((NW, NCHUNK * CAPB, SBE), jnp.int32),
            jax.ShapeDtypeStruct((NW, NCHUNK * CAPB, SBE), jnp.int32),
            jax.ShapeDtypeStruct((NW, 1, 128), jnp.int32),
        ],
        mesh=mesh,
        compiler_params=pltpu.CompilerParams(needs_layout_passes=False),
        scratch_types=[
            pltpu.VMEM((stage,), jnp.int32),
            pltpu.VMEM((NCHUNK * CAPB, SBE), jnp.int32),
            pltpu.VMEM((NCHUNK * CAPB, SBE), jnp.int32),
            pltpu.VMEM((128,), jnp.int32),
            pltpu.VMEM((1, 128), jnp.int32),
        ],
    )
    def bk(ji_hbm, bt_hbm, bj_hbm, cnt_hbm, jibuf, bt, bj, cnt, cout):
        wid = lax.axis_index("s") * NC + lax.axis_index("c")
        base = wid * PERW_T
        iota = lax.iota(jnp.int32, 16)
        zero16 = jnp.zeros((16,), jnp.int32)

        def zc(i, carry):
            cnt[pl.ds(i * 16, 16)] = zero16
            return carry
        lax.fori_loop(0, 128 // 16, zc, 0)

        def stage_body(s, carry):
            pltpu.sync_copy(ji_hbm.at[pl.ds(base + s * stage, stage)], jibuf)

            lane0 = iota == 0

            def item(i, carry2):
                ji = plsc.load_gather(jibuf, [jnp.full((16,), i, jnp.int32)])
                c = lax.shift_right_logical(ji, 13)
                p = plsc.load_gather(cnt, [c])
                f = c * CAP + jnp.minimum(p, CAP - 1)
                fh = lax.shift_right_logical(f, 7)
                fl = f & (SBE - 1)
                tid = jnp.full((16,), base + s * stage + i, jnp.int32)
                plsc.store_scatter(bt, [fh, fl], tid, mask=lane0)
                plsc.store_scatter(bj, [fh, fl], ji, mask=lane0)
                plsc.addupdate_scatter(cnt, [c], jnp.ones((16,), jnp.int32),
                                       mask=lane0)
                return carry2
            lax.fori_loop(0, stage, item, 0)
            return carry
        lax.fori_loop(0, stages, stage_body, 0)

        def padc(c, carry):
            cvec = jnp.full((16,), c, jnp.int32)
            p = jnp.minimum(jnp.min(plsc.load_gather(cnt, [cvec])), CAP)
            p2 = jnp.minimum(((p + SBE - 1) // SBE) * SBE, CAP)

            def padi(t, carry2):
                q = c * CAP + p + t * 16 + iota
                m = q < c * CAP + p2
                qh = lax.shift_right_logical(q, 7)
                ql = q & (SBE - 1)
                plsc.store_scatter(
                    bt, [qh, ql],
                    wid * 997 + c * 131 + t * 16 + iota, mask=m)
                plsc.store_scatter(
                    bj, [qh, ql],
                    jnp.full((16,), c * CH + CH + (wid & (GROWS - 1)),
                             jnp.int32), mask=m)
                return carry2
            lax.fori_loop(0, (SBE + 15) // 16, padi, 0)
            plsc.store_scatter(cnt, [jnp.full((16,), c, jnp.int32)],
                               jnp.full((16,), p2, jnp.int32),
                               mask=iota == 0)
            return carry
        lax.fori_loop(0, NCHUNK, padc, 0)

        def cw(i, carry):
            cout[0, pl.ds(i * 16, 16)] = cnt[pl.ds(i * 16, 16)]
            return carry
        lax.fori_loop(0, 128 // 16, cw, 0)

        pltpu.sync_copy(bt, bt_hbm.at[wid])
        pltpu.sync_copy(bj, bj_hbm.at[wid])
        pltpu.sync_copy(cout, cnt_hbm.at[wid])

    return bk(idx_ji)


def _sc_agg(v_att, att8, bins_tid, bins_ji, counts):
    stripe = CH // NS
    zeros = jnp.zeros((stripe, HID), jnp.float32)
    zeros8 = jnp.zeros((stripe, HEADS), jnp.float32)
    mesh = plsc.VectorSubcoreMesh(**_SC_MESH)

    @functools.partial(
        pl.kernel,
        out_type=[jax.ShapeDtypeStruct((E, HID), jnp.float32),
                  jax.ShapeDtypeStruct((E, HEADS), jnp.float32)],
        mesh=mesh,
        compiler_params=pltpu.CompilerParams(needs_layout_passes=False,
                                             use_tc_tiling_on_sc=False),
        scratch_types=[
            pltpu.VMEM_SHARED((CH + GROWS, HID), jnp.float32),
            pltpu.VMEM_SHARED((CH + GROWS, HEADS), jnp.float32),
            pltpu.VMEM((NW, 1, 128), jnp.int32),
            pltpu.VMEM((SBE,), jnp.int32),
            pltpu.VMEM((SBE,), jnp.int32),
            pltpu.VMEM((SBE,), jnp.int32),
            pltpu.VMEM((SBE, HID), jnp.float32),
            pltpu.VMEM((SBE, HEADS), jnp.float32),
            pltpu.SemaphoreType.DMA,
            pltpu.SemaphoreType.DMA,
        ],
    )
    def ek(vatt_hbm, att_hbm, bt_hbm, bj_hbm, cnt_hbm, z_hbm, z16_hbm,
           out_hbm, att_out_hbm,
           slab, aslab, cbuf, tid_v, ji_v, rel_v, rows_v, arows_v,
           sem, asem):
        cid = lax.axis_index("c")
        sid = lax.axis_index("s")
        pltpu.sync_copy(cnt_hbm, cbuf)

        def chunk_body(cc, carry):
            c = cc * NC + cid
            cbase = c * CH

            pltpu.sync_copy(z_hbm, slab.at[pl.ds(sid * stripe, stripe)])
            pltpu.sync_copy(z16_hbm, aslab.at[pl.ds(sid * stripe, stripe)])

            @pl.when(sid == 0)
            def _():
                pltpu.sync_copy(z_hbm.at[pl.ds(0, GROWS)],
                                slab.at[pl.ds(CH, GROWS)])
                pltpu.sync_copy(z16_hbm.at[pl.ds(0, GROWS)],
                                aslab.at[pl.ds(CH, GROWS)])
            plsc.subcore_barrier()

            def drain(wo, carry2):
                w = sid * 2 + wo
                npad = jnp.min(plsc.load_gather(
                    cbuf, [jnp.full((16,), w, jnp.int32),
                           jnp.zeros((16,), jnp.int32),
                           jnp.full((16,), c, jnp.int32)]))
                nb = lax.shift_right_logical(npad, 7)

                def batch(b, carry3):
                    pltpu.sync_copy(bt_hbm.at[w].at[c * CAPB + b], tid_v)
                    pltpu.sync_copy(bj_hbm.at[w].at[c * CAPB + b], ji_v)

                    def torel(i, carry4):
                        rel_v[pl.ds(i * 16, 16)] = (
                            ji_v[pl.ds(i * 16, 16)] - cbase)
                        return carry4
                    lax.fori_loop(0, SBE // 16, torel, 0)
                    cp1 = pltpu.async_copy(vatt_hbm.at[tid_v], rows_v, sem)
                    cp2 = pltpu.async_copy(att_hbm.at[tid_v], arows_v, asem)
                    cp1.wait()
                    cp2.wait()
                    pltpu.sync_copy(rows_v, slab.at[rel_v], add=True)
                    pltpu.sync_copy(arows_v, aslab.at[rel_v], add=True)
                    return carry3
                lax.fori_loop(0, nb, batch, 0)
                return carry2
            lax.fori_loop(0, 2, drain, 0)
            plsc.subcore_barrier()

            rbase = cbase + sid * stripe

            @pl.when(rbase < E)
            def _():
                pltpu.sync_copy(slab.at[pl.ds(sid * stripe, stripe)],
                                out_hbm.at[pl.ds(rbase, stripe)])
                pltpu.sync_copy(aslab.at[pl.ds(sid * stripe, stripe)],
                                att_out_hbm.at[pl.ds(rbase, stripe)])
            return carry

        lax.fori_loop(0, NCHUNK // NC, chunk_body, 0)

    return ek(v_att, att8, bins_tid, bins_ji, counts, zeros, zeros8)


def _pad_rows(x, mult=8):
    pad = (-x.shape[0]) % mult
    return jnp.pad(x, ((0, pad), (0, 0))) if pad else x


def kernel(atom_feature, edge_feature, src, dst, idx_kj, idx_ji, W_i,
           Wv0, Wk0, Wq0, r1w0, r1b0, r2w0, r2b0,
           Wv1, Wk1, Wq1, r1w1, r1b1, r2w1, r2b1,
           W_o, b_o):
    AF = atom_feature.shape[1]

    anode = _mm(atom_feature, W_i[:AF])
    feats = _mm(jnp.pad(edge_feature, ((0, 0), (0, 2))),
                _pad_rows(W_i[AF:]))
    feats = _relu(_sc_gather(anode, src) + feats)
    bins_tid, bins_ji, counts = _sc_bin(idx_ji)

    layers = [(Wv0, Wk0, Wq0, r1w0, r1b0, r2w0, r2b0),
              (Wv1, Wk1, Wq1, r1w1, r1b1, r2w1, r2b1)]
    for (Wv, Wk, Wq, r1w, r1b, r2w, r2b) in layers:
        q, k, v = _mm3(feats, Wq, Wk, Wv)
        qg = _sc_gather(q, idx_kj)
        kg = _sc_gather(k, idx_ji)
        vg = _sc_gather(v, idx_kj)
        att, v_att = _att_vatt(qg, kg, vg)
        agg, att_all = _sc_agg(v_att, att, bins_tid, bins_ji, counts)
        feats = _residual(agg, att_all, v, r1w, r1b, r2w, r2b)

    fparts = _sc_scatter_rows(feats, dst, N)
    out = _out_proj(jnp.pad(atom_feature, ((0, 0), (0, 3))),
                    fparts[0][:N], fparts[1][:N],
                    jnp.pad(W_o[:AF], ((0, 3), (0, 0))), W_o[AF:], b_o)
    return out


# speedup vs baseline: 1.9127x; 1.0629x over previous
import functools

import jax
import jax.numpy as jnp
from jax import lax
from jax.experimental import pallas as pl
from jax.experimental.pallas import tpu as pltpu
from jax.experimental.pallas import tpu_sc as plsc

N = 10000
E = 320000
T = 640000
HID = 128
HEADS = 8
DH = HID // HEADS

NC = 2
NS = 16
NW = NC * NS

_SC_MESH = dict(core_axis_name="c", subcore_axis_name="s",
                num_cores=NC, num_subcores=NS)


def _sc_gather(table, idx, sb=400):
    B = idx.shape[0]
    D = table.shape[1]
    per_w = B // NW
    assert per_w * NW == B and per_w % sb == 0 and sb % 8 == 0
    batches = per_w // sb
    if batches % 2:
        sb //= 2
        batches *= 2
    assert batches % 2 == 0 and sb % 8 == 0
    mesh = plsc.VectorSubcoreMesh(**_SC_MESH)

    @functools.partial(
        pl.kernel,
        out_type=jax.ShapeDtypeStruct((B, D), jnp.float32),
        mesh=mesh,
        scratch_types=[
            pltpu.VMEM((per_w,), jnp.int32),
            pltpu.VMEM((sb, D), jnp.float32),
            pltpu.VMEM((sb, D), jnp.float32),
            pltpu.SemaphoreType.DMA,
            pltpu.SemaphoreType.DMA,
        ],
    )
    def gk(table_hbm, idx_hbm, out_hbm, idx_v, rows0, rows1, sem0, sem1):
        wid = lax.axis_index("s") * NC + lax.axis_index("c")
        base = wid * per_w
        pltpu.sync_copy(idx_hbm.at[pl.ds(base, per_w)], idx_v)

        def gat(b, buf, sem):
            return pltpu.make_async_copy(
                table_hbm.at[idx_v.at[pl.ds(b * sb, sb)]], buf, sem)

        pltpu.async_copy(table_hbm.at[idx_v.at[pl.ds(0, sb)]], rows0, sem0)

        def body(j, carry):
            b0 = j * 2
            b1 = b0 + 1
            pltpu.async_copy(
                table_hbm.at[idx_v.at[pl.ds(b1 * sb, sb)]], rows1, sem1)
            gat(b0, rows0, sem0).wait()
            pltpu.sync_copy(rows0, out_hbm.at[pl.ds(base + b0 * sb, sb)])

            @pl.when(b0 + 2 < batches)
            def _():
                pltpu.async_copy(
                    table_hbm.at[idx_v.at[pl.ds((b0 + 2) * sb, sb)]],
                    rows0, sem0)
            gat(b1, rows1, sem1).wait()
            pltpu.sync_copy(rows1, out_hbm.at[pl.ds(base + b1 * sb, sb)])
            return carry

        lax.fori_loop(0, batches // 2, body, 0)

    return gk(table, idx)


def _relu(x):
    return jnp.maximum(x, 0.0)


def _leaky(x):
    return jnp.where(x >= 0, x, 0.2 * x)


def _mm_kernel(x_ref, w_ref, b_ref, o_ref, *, act):
    x = x_ref[...]
    w = w_ref[...]
    y = jax.lax.dot_general(x, w, (((1,), (0,)), ((), ())),
                            preferred_element_type=jnp.float32)
    y = y + b_ref[...]
    if act == "relu":
        y = jnp.maximum(y, 0.0)
    o_ref[...] = y


def _mm(x, w, b=None, act="none", block_rows=512):
    R, K = x.shape
    Kw, Nout = w.shape
    assert K == Kw
    if b is None:
        b = jnp.zeros((Nout,), dtype=jnp.float32)
    pad_r = (-R) % block_rows
    if pad_r:
        x = jnp.pad(x, ((0, pad_r), (0, 0)))
    Rp = R + pad_r
    grid = (Rp // block_rows,)
    out = pl.pallas_call(
        functools.partial(_mm_kernel, act=act),
        grid=grid,
        in_specs=[
            pl.BlockSpec((block_rows, K), lambda i: (i, 0)),
            pl.BlockSpec((K, Nout), lambda i: (0, 0)),
            pl.BlockSpec((Nout,), lambda i: (0,)),
        ],
        out_specs=pl.BlockSpec((block_rows, Nout), lambda i: (i, 0)),
        out_shape=jax.ShapeDtypeStruct((Rp, Nout), jnp.float32),
    )(x, w, b)
    return out[:R] if pad_r else out


def _mm3_kernel(x_ref, w_ref, o1_ref, o2_ref, o3_ref):
    y = jax.lax.dot_general(x_ref[...], w_ref[...], (((1,), (0,)), ((), ())),
                            preferred_element_type=jnp.float32)
    o1_ref[...] = y[:, :HID]
    o2_ref[...] = y[:, HID:2 * HID]
    o3_ref[...] = y[:, 2 * HID:]


def _mm3(x, w1, w2, w3, block_rows=1000):
    R, K = x.shape
    w = jnp.concatenate([w1, w2, w3], axis=1)
    assert R % block_rows == 0
    grid = (R // block_rows,)
    return pl.pallas_call(
        _mm3_kernel,
        grid=grid,
        in_specs=[
            pl.BlockSpec((block_rows, K), lambda i: (i, 0)),
            pl.BlockSpec((K, 3 * HID), lambda i: (0, 0)),
        ],
        out_specs=[pl.BlockSpec((block_rows, HID), lambda i: (i, 0))] * 3,
        out_shape=[jax.ShapeDtypeStruct((R, HID), jnp.float32)] * 3,
    )(x, w)


def _res_kernel(x_ref, d_ref, v_ref, st_ref, w1_ref, b1_ref, w2_ref, b2_ref,
                o_ref):
    den = jax.lax.dot_general(d_ref[...], st_ref[...],
                              (((1,), (0,)), ((), ())),
                              preferred_element_type=jnp.float32)
    x = x_ref[...] / jnp.maximum(den, 1e-30)
    h = jax.lax.dot_general(x, w1_ref[...], (((1,), (0,)), ((), ())),
                            preferred_element_type=jnp.float32)
    h = jnp.maximum(h + b1_ref[...], 0.0)
    y = jax.lax.dot_general(h, w2_ref[...], (((1,), (0,)), ((), ())),
                            preferred_element_type=jnp.float32)
    o_ref[...] = v_ref[...] + jnp.maximum(y + b2_ref[...], 0.0)


def _residual(x, att_all, vflat, w1, b1, w2, b2, block_rows=1000):
    R, K = x.shape
    assert R % block_rows == 0
    sel = (jnp.arange(HID)[:, None] // DH
           == jnp.arange(HEADS)[None, :]).astype(jnp.float32)
    grid = (R // block_rows,)
    return pl.pallas_call(
        _res_kernel,
        grid=grid,
        in_specs=[
            pl.BlockSpec((block_rows, K), lambda i: (i, 0)),
            pl.BlockSpec((block_rows, HEADS), lambda i: (i, 0)),
            pl.BlockSpec((block_rows, HID), lambda i: (i, 0)),
            pl.BlockSpec((HEADS, HID), lambda i: (0, 0)),
            pl.BlockSpec((K, HID), lambda i: (0, 0)),
            pl.BlockSpec((HID,), lambda i: (0,)),
            pl.BlockSpec((HID, HID), lambda i: (0, 0)),
            pl.BlockSpec((HID,), lambda i: (0,)),
        ],
        out_specs=pl.BlockSpec((block_rows, HID), lambda i: (i, 0)),
        out_shape=jax.ShapeDtypeStruct((R, HID), jnp.float32),
    )(x, att_all, vflat, sel.T, w1, b1, w2, b2)


def _att_kernel(q_ref, k_ref, v_ref, s_ref, st_ref, a_ref, va_ref):
    qk = q_ref[...] * k_ref[...]
    a = jax.lax.dot_general(qk, s_ref[...], (((1,), (0,)), ((), ())),
                            preferred_element_type=jnp.float32)
    a = jnp.exp(jnp.where(a >= 0, a, 0.2 * a))
    a_ref[...] = a
    ab = jax.lax.dot_general(a, st_ref[...], (((1,), (0,)), ((), ())),
                             preferred_element_type=jnp.float32)
    va_ref[...] = v_ref[...] * ab


def _att_vatt(qg, kg, vg, block_rows=1000):
    R = qg.shape[0]
    assert R % block_rows == 0
    sel = (jnp.arange(HID)[:, None] // DH
           == jnp.arange(HEADS)[None, :]).astype(jnp.float32)
    grid = (R // block_rows,)
    return pl.pallas_call(
        _att_kernel,
        grid=grid,
        in_specs=[
            pl.BlockSpec((block_rows, HID), lambda i: (i, 0)),
            pl.BlockSpec((block_rows, HID), lambda i: (i, 0)),
            pl.BlockSpec((block_rows, HID), lambda i: (i, 0)),
            pl.BlockSpec((HID, HEADS), lambda i: (0, 0)),
            pl.BlockSpec((HEADS, HID), lambda i: (0, 0)),
        ],
        out_specs=[pl.BlockSpec((block_rows, HEADS), lambda i: (i, 0)),
                   pl.BlockSpec((block_rows, HID), lambda i: (i, 0))],
        out_shape=[jax.ShapeDtypeStruct((R, HEADS), jnp.float32),
                   jax.ShapeDtypeStruct((R, HID), jnp.float32)],
    )(qg, kg, vg, sel, sel.T)


def _out_kernel(a_ref, f0_ref, f1_ref, w1_ref, w2_ref, b_ref, o_ref):
    y = jax.lax.dot_general(a_ref[...], w1_ref[...], (((1,), (0,)), ((), ())),
                            preferred_element_type=jnp.float32)
    f = f0_ref[...] + f1_ref[...]
    y = y + jax.lax.dot_general(f, w2_ref[...], (((1,), (0,)), ((), ())),
                                preferred_element_type=jnp.float32)
    o_ref[...] = jnp.maximum(y + b_ref[...], 0.0)


def _out_proj(atom, f0, f1, w1, w2, b, block_rows=400):
    R, K = atom.shape
    assert R % block_rows == 0
    grid = (R // block_rows,)
    return pl.pallas_call(
        _out_kernel,
        grid=grid,
        in_specs=[
            pl.BlockSpec((block_rows, K), lambda i: (i, 0)),
            pl.BlockSpec((block_rows, HID), lambda i: (i, 0)),
            pl.BlockSpec((block_rows, HID), lambda i: (i, 0)),
            pl.BlockSpec((K, HID), lambda i: (0, 0)),
            pl.BlockSpec((HID, HID), lambda i: (0, 0)),
            pl.BlockSpec((HID,), lambda i: (0,)),
        ],
        out_specs=pl.BlockSpec((block_rows, HID), lambda i: (i, 0)),
        out_shape=jax.ShapeDtypeStruct((R, HID), jnp.float32),
    )(atom, f0, f1, w1, w2, b)


def _sc_scatter_rows(values, idx, n_out, sb=80):
    B = values.shape[0]
    per_w = B // NW
    assert per_w * NW == B and per_w % sb == 0 and sb % 8 == 0
    batches = per_w // sb
    n_pad = -(-n_out // (NS * 8)) * (NS * 8)
    per_t = n_pad // NS
    idx3d = idx.reshape(NW, batches, sb)
    zeros = jnp.zeros((per_t, HID), jnp.float32)
    mesh = plsc.VectorSubcoreMesh(**_SC_MESH)

    @functools.partial(
        pl.kernel,
        out_type=jax.ShapeDtypeStruct((NC, n_pad, HID), jnp.float32),
        mesh=mesh,
        scratch_types=[
            pltpu.VMEM_SHARED((n_pad, HID), jnp.float32),
            pltpu.VMEM((sb,), jnp.int32),
            pltpu.VMEM((sb, HID), jnp.float32),
            pltpu.SemaphoreType.DMA,
        ],
    )
    def sk(val_hbm, idx_hbm, z_hbm, out_hbm, slab, idx_v, rows_v, sem):
        cid = lax.axis_index("c")
        sid = lax.axis_index("s")
        wid = sid * NC + cid
        base = wid * per_w

        pltpu.sync_copy(z_hbm, slab.at[pl.ds(sid * per_t, per_t)])
        plsc.subcore_barrier()

        def body(j, carry):
            pltpu.sync_copy(idx_hbm.at[wid].at[j], idx_v)
            pltpu.async_copy(val_hbm.at[pl.ds(base + j * sb, sb)],
                             rows_v, sem).wait()
            pltpu.sync_copy(rows_v, slab.at[idx_v], add=True)
            return carry
        lax.fori_loop(0, batches, body, 0)

        plsc.subcore_barrier()
        pltpu.sync_copy(slab.at[pl.ds(sid * per_t, per_t)],
                        out_hbm.at[cid].at[pl.ds(sid * per_t, per_t)])

    return sk(values, idx3d, zeros)


CH = 8192
NCHUNK = -(-E // CH)
CAP = 768
SBE = 128
CAPB = CAP // SBE
GROWS = 16
PERW_T = T // NW


def _sc_bin(idx_ji):
    stage = 2000
    stages = PERW_T // stage
    mesh = plsc.VectorSubcoreMesh(**_SC_MESH)

    @functools.partial(
        pl.kernel,
        out_type=[
            jax.ShapeDtypeStruct((NW, NCHUNK * CAPB, SBE), jnp.int32),
            jax.ShapeDtypeStruct((NW, NCHUNK * CAPB, SBE), jnp.int32),
            jax.ShapeDtypeStruct((NW, 1, 128), jnp.int32),
        ],
        mesh=mesh,
        compiler_params=pltpu.CompilerParams(needs_layout_passes=False),
        scratch_types=[
            pltpu.VMEM((stage,), jnp.int32),
            pltpu.VMEM((NCHUNK * CAPB, SBE), jnp.int32),
            pltpu.VMEM((NCHUNK * CAPB, SBE), jnp.int32),
            pltpu.VMEM((128,), jnp.int32),
            pltpu.VMEM((1, 128), jnp.int32),
        ],
    )
    def bk(ji_hbm, bt_hbm, bj_hbm, cnt_hbm, jibuf, bt, bj, cnt, cout):
        wid = lax.axis_index("s") * NC + lax.axis_index("c")
        base = wid * PERW_T
        iota = lax.iota(jnp.int32, 16)
        zero16 = jnp.zeros((16,), jnp.int32)

        def zc(i, carry):
            cnt[pl.ds(i * 16, 16)] = zero16
            return carry
        lax.fori_loop(0, 128 // 16, zc, 0)

        def stage_body(s, carry):
            pltpu.sync_copy(ji_hbm.at[pl.ds(base + s * stage, stage)], jibuf)

            lane0 = iota == 0

            def item(i, carry2):
                ji = plsc.load_gather(jibuf, [jnp.full((16,), i, jnp.int32)])
                c = lax.shift_right_logical(ji, 13)
                p = plsc.load_gather(cnt, [c])
                f = c * CAP + jnp.minimum(p, CAP - 1)
                fh = lax.shift_right_logical(f, 7)
                fl = f & (SBE - 1)
                tid = jnp.full((16,), base + s * stage + i, jnp.int32)
                plsc.store_scatter(bt, [fh, fl], tid, mask=lane0)
                plsc.store_scatter(bj, [fh, fl], ji, mask=lane0)
                plsc.addupdate_scatter(cnt, [c], jnp.ones((16,), jnp.int32),
                                       mask=lane0)
                return carry2
            lax.fori_loop(0, stage, item, 0)
            return carry
        lax.fori_loop(0, stages, stage_body, 0)

        def padc(c, carry):
            cvec = jnp.full((16,), c, jnp.int32)
            p = jnp.minimum(jnp.min(plsc.load_gather(cnt, [cvec])), CAP)
            p2 = jnp.minimum(((p + SBE - 1) // SBE) * SBE, CAP)

            def padi(t, carry2):
                q = c * CAP + p + t * 16 + iota
                m = q < c * CAP + p2
                qh = lax.shift_right_logical(q, 7)
                ql = q & (SBE - 1)
                plsc.store_scatter(
                    bt, [qh, ql],
                    wid * 997 + c * 131 + t * 16 + iota, mask=m)
                plsc.store_scatter(
                    bj, [qh, ql],
                    jnp.full((16,), c * CH + CH + (wid & (GROWS - 1)),
                             jnp.int32), mask=m)
                return carry2
            lax.fori_loop(0, (SBE + 15) // 16, padi, 0)
            plsc.store_scatter(cnt, [jnp.full((16,), c, jnp.int32)],
                               jnp.full((16,), p2, jnp.int32),
                               mask=iota == 0)
            return carry
        lax.fori_loop(0, NCHUNK, padc, 0)

        def cw(i, carry):
            cout[0, pl.ds(i * 16, 16)] = cnt[pl.ds(i * 16, 16)]
            return carry
        lax.fori_loop(0, 128 // 16, cw, 0)

        pltpu.sync_copy(bt, bt_hbm.at[wid])
        pltpu.sync_copy(bj, bj_hbm.at[wid])
        pltpu.sync_copy(cout, cnt_hbm.at[wid])

    return bk(idx_ji)


def _sc_agg(v_att, att8, bins_tid, bins_ji, counts):
    stripe = CH // NS
    zeros = jnp.zeros((stripe, HID), jnp.float32)
    zeros8 = jnp.zeros((stripe, HEADS), jnp.float32)
    mesh = plsc.VectorSubcoreMesh(**_SC_MESH)

    @functools.partial(
        pl.kernel,
        out_type=[jax.ShapeDtypeStruct((E, HID), jnp.float32),
                  jax.ShapeDtypeStruct((E, HEADS), jnp.float32)],
        mesh=mesh,
        compiler_params=pltpu.CompilerParams(needs_layout_passes=False,
                                             use_tc_tiling_on_sc=False),
        scratch_types=[
            pltpu.VMEM_SHARED((CH + GROWS, HID), jnp.float32),
            pltpu.VMEM_SHARED((CH + GROWS, HEADS), jnp.float32),
            pltpu.VMEM((NW, 1, 128), jnp.int32),
            pltpu.VMEM((SBE,), jnp.int32),
            pltpu.VMEM((SBE,), jnp.int32),
            pltpu.VMEM((SBE,), jnp.int32),
            pltpu.VMEM((SBE, HID), jnp.float32),
            pltpu.VMEM((SBE, HEADS), jnp.float32),
            pltpu.VMEM((SBE,), jnp.int32),
            pltpu.VMEM((SBE,), jnp.int32),
            pltpu.VMEM((SBE,), jnp.int32),
            pltpu.VMEM((SBE, HID), jnp.float32),
            pltpu.VMEM((SBE, HEADS), jnp.float32),
            pltpu.SemaphoreType.DMA,
            pltpu.SemaphoreType.DMA,
            pltpu.SemaphoreType.DMA,
            pltpu.SemaphoreType.DMA,
        ],
    )
    def ek(vatt_hbm, att_hbm, bt_hbm, bj_hbm, cnt_hbm, z_hbm, z16_hbm,
           out_hbm, att_out_hbm,
           slab, aslab, cbuf, tid_v, ji_v, rel_v, rows_v, arows_v,
           tid2_v, ji2_v, rel2_v, rows2_v, arows2_v,
           sem, asem, sem2, asem2):
        cid = lax.axis_index("c")
        sid = lax.axis_index("s")
        pltpu.sync_copy(cnt_hbm, cbuf)

        def chunk_body(cc, carry):
            c = cc * NC + cid
            cbase = c * CH

            pltpu.sync_copy(z_hbm, slab.at[pl.ds(sid * stripe, stripe)])
            pltpu.sync_copy(z16_hbm, aslab.at[pl.ds(sid * stripe, stripe)])

            @pl.when(sid == 0)
            def _():
                pltpu.sync_copy(z_hbm.at[pl.ds(0, GROWS)],
                                slab.at[pl.ds(CH, GROWS)])
                pltpu.sync_copy(z16_hbm.at[pl.ds(0, GROWS)],
                                aslab.at[pl.ds(CH, GROWS)])
            plsc.subcore_barrier()

            w0 = sid * 2
            npad0 = jnp.min(plsc.load_gather(
                cbuf, [jnp.full((16,), w0, jnp.int32),
                       jnp.zeros((16,), jnp.int32),
                       jnp.full((16,), c, jnp.int32)]))
            npad1 = jnp.min(plsc.load_gather(
                cbuf, [jnp.full((16,), w0 + 1, jnp.int32),
                       jnp.zeros((16,), jnp.int32),
                       jnp.full((16,), c, jnp.int32)]))
            nb0 = lax.shift_right_logical(npad0, 7)
            nb = nb0 + lax.shift_right_logical(npad1, 7)

            def row_of(b):
                w = jnp.where(b < nb0, w0, w0 + 1)
                r = jnp.where(b < nb0, b, b - nb0) + c * CAPB
                return w, r

            def fire(b, tid_b, ji_b, rel_b, rows_b, arows_b, sem_b, asem_b):
                w, r = row_of(b)
                pltpu.sync_copy(bt_hbm.at[w].at[r], tid_b)
                pltpu.sync_copy(bj_hbm.at[w].at[r], ji_b)

                def torel(i, carry4):
                    rel_b[pl.ds(i * 16, 16)] = (
                        ji_b[pl.ds(i * 16, 16)] - cbase)
                    return carry4
                lax.fori_loop(0, SBE // 16, torel, 0)
                pltpu.async_copy(vatt_hbm.at[tid_b], rows_b, sem_b)
                pltpu.async_copy(att_hbm.at[tid_b], arows_b, asem_b)

            def drain_sc(tid_b, rel_b, rows_b, arows_b, sem_b, asem_b):
                pltpu.make_async_copy(vatt_hbm.at[tid_b], rows_b,
                                      sem_b).wait()
                pltpu.make_async_copy(att_hbm.at[tid_b], arows_b,
                                      asem_b).wait()
                pltpu.sync_copy(rows_b, slab.at[rel_b], add=True)
                pltpu.sync_copy(arows_b, aslab.at[rel_b], add=True)

            @pl.when(nb > 0)
            def _():
                fire(0, tid_v, ji_v, rel_v, rows_v, arows_v, sem, asem)

            def batch2(j, carry3):
                b0 = j * 2
                b1 = b0 + 1
                fire(b1, tid2_v, ji2_v, rel2_v, rows2_v, arows2_v,
                     sem2, asem2)
                drain_sc(tid_v, rel_v, rows_v, arows_v, sem, asem)

                @pl.when(b0 + 2 < nb)
                def _():
                    fire(b0 + 2, tid_v, ji_v, rel_v, rows_v, arows_v,
                         sem, asem)
                drain_sc(tid2_v, rel2_v, rows2_v, arows2_v, sem2, asem2)
                return carry3
            lax.fori_loop(0, lax.shift_right_logical(nb, 1), batch2, 0)

            @pl.when(nb & 1 == 1)
            def _():
                drain_sc(tid_v, rel_v, rows_v, arows_v, sem, asem)
            plsc.subcore_barrier()

            rbase = cbase + sid * stripe

            @pl.when(rbase < E)
            def _():
                pltpu.sync_copy(slab.at[pl.ds(sid * stripe, stripe)],
                                out_hbm.at[pl.ds(rbase, stripe)])
                pltpu.sync_copy(aslab.at[pl.ds(sid * stripe, stripe)],
                                att_out_hbm.at[pl.ds(rbase, stripe)])
            return carry

        lax.fori_loop(0, NCHUNK // NC, chunk_body, 0)

    return ek(v_att, att8, bins_tid, bins_ji, counts, zeros, zeros8)


def _pad_rows(x, mult=8):
    pad = (-x.shape[0]) % mult
    return jnp.pad(x, ((0, pad), (0, 0))) if pad else x


def kernel(atom_feature, edge_feature, src, dst, idx_kj, idx_ji, W_i,
           Wv0, Wk0, Wq0, r1w0, r1b0, r2w0, r2b0,
           Wv1, Wk1, Wq1, r1w1, r1b1, r2w1, r2b1,
           W_o, b_o):
    AF = atom_feature.shape[1]

    anode = _mm(atom_feature, W_i[:AF])
    feats = _mm(jnp.pad(edge_feature, ((0, 0), (0, 2))),
                _pad_rows(W_i[AF:]))
    feats = _relu(_sc_gather(anode, src) + feats)
    bins_tid, bins_ji, counts = _sc_bin(idx_ji)

    layers = [(Wv0, Wk0, Wq0, r1w0, r1b0, r2w0, r2b0),
              (Wv1, Wk1, Wq1, r1w1, r1b1, r2w1, r2b1)]
    for (Wv, Wk, Wq, r1w, r1b, r2w, r2b) in layers:
        q, k, v = _mm3(feats, Wq, Wk, Wv)
        qg = _sc_gather(q, idx_kj)
        kg = _sc_gather(k, idx_ji)
        vg = _sc_gather(v, idx_kj)
        att, v_att = _att_vatt(qg, kg, vg)
        agg, att_all = _sc_agg(v_att, att, bins_tid, bins_ji, counts)
        feats = _residual(agg, att_all, v, r1w, r1b, r2w, r2b)

    fparts = _sc_scatter_rows(feats, dst, N)
    out = _out_proj(jnp.pad(atom_feature, ((0, 0), (0, 3))),
                    fparts[0][:N], fparts[1][:N],
                    jnp.pad(W_o[:AF], ((0, 3), (0, 0))), W_o[AF:], b_o)
    return out


# speedup vs baseline: 1.9128x; 1.0001x over previous
import functools

import jax
import jax.numpy as jnp
from jax import lax
from jax.experimental import pallas as pl
from jax.experimental.pallas import tpu as pltpu
from jax.experimental.pallas import tpu_sc as plsc

N = 10000
E = 320000
T = 640000
HID = 128
HEADS = 8
DH = HID // HEADS

NC = 2
NS = 16
NW = NC * NS

_SC_MESH = dict(core_axis_name="c", subcore_axis_name="s",
                num_cores=NC, num_subcores=NS)


def _sc_gather(table, idx, sb=400):
    B = idx.shape[0]
    D = table.shape[1]
    per_w = B // NW
    assert per_w * NW == B and per_w % sb == 0 and sb % 8 == 0
    batches = per_w // sb
    if batches % 2:
        sb //= 2
        batches *= 2
    assert batches % 2 == 0 and sb % 8 == 0
    mesh = plsc.VectorSubcoreMesh(**_SC_MESH)

    @functools.partial(
        pl.kernel,
        out_type=jax.ShapeDtypeStruct((B, D), jnp.float32),
        mesh=mesh,
        scratch_types=[
            pltpu.VMEM((per_w,), jnp.int32),
            pltpu.VMEM((sb, D), jnp.float32),
            pltpu.VMEM((sb, D), jnp.float32),
            pltpu.SemaphoreType.DMA,
            pltpu.SemaphoreType.DMA,
        ],
    )
    def gk(table_hbm, idx_hbm, out_hbm, idx_v, rows0, rows1, sem0, sem1):
        wid = lax.axis_index("s") * NC + lax.axis_index("c")
        base = wid * per_w
        pltpu.sync_copy(idx_hbm.at[pl.ds(base, per_w)], idx_v)

        def gat(b, buf, sem):
            return pltpu.make_async_copy(
                table_hbm.at[idx_v.at[pl.ds(b * sb, sb)]], buf, sem)

        pltpu.async_copy(table_hbm.at[idx_v.at[pl.ds(0, sb)]], rows0, sem0)

        def body(j, carry):
            b0 = j * 2
            b1 = b0 + 1
            pltpu.async_copy(
                table_hbm.at[idx_v.at[pl.ds(b1 * sb, sb)]], rows1, sem1)
            gat(b0, rows0, sem0).wait()
            pltpu.sync_copy(rows0, out_hbm.at[pl.ds(base + b0 * sb, sb)])

            @pl.when(b0 + 2 < batches)
            def _():
                pltpu.async_copy(
                    table_hbm.at[idx_v.at[pl.ds((b0 + 2) * sb, sb)]],
                    rows0, sem0)
            gat(b1, rows1, sem1).wait()
            pltpu.sync_copy(rows1, out_hbm.at[pl.ds(base + b1 * sb, sb)])
            return carry

        lax.fori_loop(0, batches // 2, body, 0)

    return gk(table, idx)


def _relu(x):
    return jnp.maximum(x, 0.0)


def _mm_kernel(x_ref, w_ref, b_ref, o_ref, *, act):
    x = x_ref[...]
    w = w_ref[...]
    y = jax.lax.dot_general(x, w, (((1,), (0,)), ((), ())),
                            preferred_element_type=jnp.float32)
    y = y + b_ref[...]
    if act == "relu":
        y = jnp.maximum(y, 0.0)
    o_ref[...] = y


def _mm(x, w, b=None, act="none", block_rows=512):
    R, K = x.shape
    Kw, Nout = w.shape
    assert K == Kw
    if b is None:
        b = jnp.zeros((Nout,), dtype=jnp.float32)
    pad_r = (-R) % block_rows
    if pad_r:
        x = jnp.pad(x, ((0, pad_r), (0, 0)))
    Rp = R + pad_r
    grid = (Rp // block_rows,)
    out = pl.pallas_call(
        functools.partial(_mm_kernel, act=act),
        grid=grid,
        in_specs=[
            pl.BlockSpec((block_rows, K), lambda i: (i, 0)),
            pl.BlockSpec((K, Nout), lambda i: (0, 0)),
            pl.BlockSpec((Nout,), lambda i: (0,)),
        ],
        out_specs=pl.BlockSpec((block_rows, Nout), lambda i: (i, 0)),
        out_shape=jax.ShapeDtypeStruct((Rp, Nout), jnp.float32),
    )(x, w, b)
    return out[:R] if pad_r else out


def _mm3_kernel(x_ref, w_ref, o1_ref, o2_ref, o3_ref):
    y = jax.lax.dot_general(x_ref[...], w_ref[...], (((1,), (0,)), ((), ())),
                            preferred_element_type=jnp.float32)
    o1_ref[...] = y[:, :HID]
    o2_ref[...] = y[:, HID:2 * HID]
    o3_ref[...] = y[:, 2 * HID:]


def _mm3(x, w1, w2, w3, block_rows=1000):
    R, K = x.shape
    w = jnp.concatenate([w1, w2, w3], axis=1)
    assert R % block_rows == 0
    grid = (R // block_rows,)
    return pl.pallas_call(
        _mm3_kernel,
        grid=grid,
        in_specs=[
            pl.BlockSpec((block_rows, K), lambda i: (i, 0)),
            pl.BlockSpec((K, 3 * HID), lambda i: (0, 0)),
        ],
        out_specs=[pl.BlockSpec((block_rows, HID), lambda i: (i, 0))] * 3,
        out_shape=[jax.ShapeDtypeStruct((R, HID), jnp.float32)] * 3,
    )(x, w)


def _res_kernel(x_ref, d_ref, v_ref, st_ref, w1_ref, b1_ref, w2_ref, b2_ref,
                o_ref):
    den = jax.lax.dot_general(d_ref[...], st_ref[...],
                              (((1,), (0,)), ((), ())),
                              preferred_element_type=jnp.float32)
    x = x_ref[...] / jnp.maximum(den, 1e-30)
    h = jax.lax.dot_general(x, w1_ref[...], (((1,), (0,)), ((), ())),
                            preferred_element_type=jnp.float32)
    h = jnp.maximum(h + b1_ref[...], 0.0)
    y = jax.lax.dot_general(h, w2_ref[...], (((1,), (0,)), ((), ())),
                            preferred_element_type=jnp.float32)
    o_ref[...] = v_ref[...] + jnp.maximum(y + b2_ref[...], 0.0)


def _residual(x, att_all, vflat, w1, b1, w2, b2, block_rows=1000):
    R, K = x.shape
    assert R % block_rows == 0
    sel = (jnp.arange(HID)[:, None] // DH
           == jnp.arange(HEADS)[None, :]).astype(jnp.float32)
    grid = (R // block_rows,)
    return pl.pallas_call(
        _res_kernel,
        grid=grid,
        in_specs=[
            pl.BlockSpec((block_rows, K), lambda i: (i, 0)),
            pl.BlockSpec((block_rows, HEADS), lambda i: (i, 0)),
            pl.BlockSpec((block_rows, HID), lambda i: (i, 0)),
            pl.BlockSpec((HEADS, HID), lambda i: (0, 0)),
            pl.BlockSpec((K, HID), lambda i: (0, 0)),
            pl.BlockSpec((HID,), lambda i: (0,)),
            pl.BlockSpec((HID, HID), lambda i: (0, 0)),
            pl.BlockSpec((HID,), lambda i: (0,)),
        ],
        out_specs=pl.BlockSpec((block_rows, HID), lambda i: (i, 0)),
        out_shape=jax.ShapeDtypeStruct((R, HID), jnp.float32),
    )(x, att_all, vflat, sel.T, w1, b1, w2, b2)


def _att_kernel(q_ref, k_ref, v_ref, s_ref, st_ref, a_ref, va_ref):
    qk = q_ref[...] * k_ref[...]
    a = jax.lax.dot_general(qk, s_ref[...], (((1,), (0,)), ((), ())),
                            preferred_element_type=jnp.float32)
    a = jnp.exp(jnp.where(a >= 0, a, 0.2 * a))
    a_ref[...] = a
    ab = jax.lax.dot_general(a, st_ref[...], (((1,), (0,)), ((), ())),
                             preferred_element_type=jnp.float32)
    va_ref[...] = v_ref[...] * ab


def _att_vatt(qg, kg, vg, block_rows=1000):
    R = qg.shape[0]
    assert R % block_rows == 0
    sel = (jnp.arange(HID)[:, None] // DH
           == jnp.arange(HEADS)[None, :]).astype(jnp.float32)
    grid = (R // block_rows,)
    return pl.pallas_call(
        _att_kernel,
        grid=grid,
        in_specs=[
            pl.BlockSpec((block_rows, HID), lambda i: (i, 0)),
            pl.BlockSpec((block_rows, HID), lambda i: (i, 0)),
            pl.BlockSpec((block_rows, HID), lambda i: (i, 0)),
            pl.BlockSpec((HID, HEADS), lambda i: (0, 0)),
            pl.BlockSpec((HEADS, HID), lambda i: (0, 0)),
        ],
        out_specs=[pl.BlockSpec((block_rows, HEADS), lambda i: (i, 0)),
                   pl.BlockSpec((block_rows, HID), lambda i: (i, 0))],
        out_shape=[jax.ShapeDtypeStruct((R, HEADS), jnp.float32),
                   jax.ShapeDtypeStruct((R, HID), jnp.float32)],
    )(qg, kg, vg, sel, sel.T)


def _out_kernel(a_ref, f0_ref, f1_ref, w1_ref, w2_ref, b_ref, o_ref):
    y = jax.lax.dot_general(a_ref[...], w1_ref[...], (((1,), (0,)), ((), ())),
                            preferred_element_type=jnp.float32)
    f = f0_ref[...] + f1_ref[...]
    y = y + jax.lax.dot_general(f, w2_ref[...], (((1,), (0,)), ((), ())),
                                preferred_element_type=jnp.float32)
    o_ref[...] = jnp.maximum(y + b_ref[...], 0.0)


def _out_proj(atom, f0, f1, w1, w2, b, block_rows=400):
    R, K = atom.shape
    assert R % block_rows == 0
    grid = (R // block_rows,)
    return pl.pallas_call(
        _out_kernel,
        grid=grid,
        in_specs=[
            pl.BlockSpec((block_rows, K), lambda i: (i, 0)),
            pl.BlockSpec((block_rows, HID), lambda i: (i, 0)),
            pl.BlockSpec((block_rows, HID), lambda i: (i, 0)),
            pl.BlockSpec((K, HID), lambda i: (0, 0)),
            pl.BlockSpec((HID, HID), lambda i: (0, 0)),
            pl.BlockSpec((HID,), lambda i: (0,)),
        ],
        out_specs=pl.BlockSpec((block_rows, HID), lambda i: (i, 0)),
        out_shape=jax.ShapeDtypeStruct((R, HID), jnp.float32),
    )(atom, f0, f1, w1, w2, b)


def _sc_scatter_rows(values, idx, n_out, sb=80):
    B = values.shape[0]
    per_w = B // NW
    assert per_w * NW == B and per_w % sb == 0 and sb % 8 == 0
    batches = per_w // sb
    n_pad = -(-n_out // (NS * 8)) * (NS * 8)
    per_t = n_pad // NS
    idx3d = idx.reshape(NW, batches, sb)
    zeros = jnp.zeros((per_t, HID), jnp.float32)
    mesh = plsc.VectorSubcoreMesh(**_SC_MESH)

    @functools.partial(
        pl.kernel,
        out_type=jax.ShapeDtypeStruct((NC, n_pad, HID), jnp.float32),
        mesh=mesh,
        scratch_types=[
            pltpu.VMEM_SHARED((n_pad, HID), jnp.float32),
            pltpu.VMEM((sb,), jnp.int32),
            pltpu.VMEM((sb, HID), jnp.float32),
            pltpu.SemaphoreType.DMA,
        ],
    )
    def sk(val_hbm, idx_hbm, z_hbm, out_hbm, slab, idx_v, rows_v, sem):
        cid = lax.axis_index("c")
        sid = lax.axis_index("s")
        wid = sid * NC + cid
        base = wid * per_w

        pltpu.sync_copy(z_hbm, slab.at[pl.ds(sid * per_t, per_t)])
        plsc.subcore_barrier()

        def body(j, carry):
            pltpu.sync_copy(idx_hbm.at[wid].at[j], idx_v)
            pltpu.async_copy(val_hbm.at[pl.ds(base + j * sb, sb)],
                             rows_v, sem).wait()
            pltpu.sync_copy(rows_v, slab.at[idx_v], add=True)
            return carry
        lax.fori_loop(0, batches, body, 0)

        plsc.subcore_barrier()
        pltpu.sync_copy(slab.at[pl.ds(sid * per_t, per_t)],
                        out_hbm.at[cid].at[pl.ds(sid * per_t, per_t)])

    return sk(values, idx3d, zeros)


CH = 8192
NCHUNK = -(-E // CH)
CAP = 768
SBE = 128
CAPB = CAP // SBE
GROWS = 16
PERW_T = T // NW


def _sc_bin(idx_ji):
    stage = 2000
    stages = PERW_T // stage
    mesh = plsc.VectorSubcoreMesh(**_SC_MESH)

    @functools.partial(
        pl.kernel,
        out_type=[
            jax.ShapeDtypeStruct((NW, NCHUNK * CAPB, SBE), jnp.int32),
            jax.ShapeDtypeStruct((NW, NCHUNK * CAPB, SBE), jnp.int32),
            jax.ShapeDtypeStruct((NW, 1, 128), jnp.int32),
        ],
        mesh=mesh,
        compiler_params=pltpu.CompilerParams(needs_layout_passes=False),
        scratch_types=[
            pltpu.VMEM((stage,), jnp.int32),
            pltpu.VMEM((NCHUNK * CAPB, SBE), jnp.int32),
            pltpu.VMEM((NCHUNK * CAPB, SBE), jnp.int32),
            pltpu.VMEM((128,), jnp.int32),
            pltpu.VMEM((1, 128), jnp.int32),
        ],
    )
    def bk(ji_hbm, bt_hbm, bj_hbm, cnt_hbm, jibuf, bt, bj, cnt, cout):
        wid = lax.axis_index("s") * NC + lax.axis_index("c")
        base = wid * PERW_T
        iota = lax.iota(jnp.int32, 16)
        zero16 = jnp.zeros((16,), jnp.int32)

        def zc(i, carry):
            cnt[pl.ds(i * 16, 16)] = zero16
            return carry
        lax.fori_loop(0, 128 // 16, zc, 0)

        def stage_body(s, carry):
            pltpu.sync_copy(ji_hbm.at[pl.ds(base + s * stage, stage)], jibuf)

            lane0 = iota == 0

            def item(i, carry2):
                ji = plsc.load_gather(jibuf, [jnp.full((16,), i, jnp.int32)])
                c = lax.shift_right_logical(ji, 13)
                p = plsc.load_gather(cnt, [c])
                f = c * CAP + jnp.minimum(p, CAP - 1)
                fh = lax.shift_right_logical(f, 7)
                fl = f & (SBE - 1)
                tid = jnp.full((16,), base + s * stage + i, jnp.int32)
                plsc.store_scatter(bt, [fh, fl], tid, mask=lane0)
                plsc.store_scatter(bj, [fh, fl], ji, mask=lane0)
                plsc.addupdate_scatter(cnt, [c], jnp.ones((16,), jnp.int32),
                                       mask=lane0)
                return carry2
            lax.fori_loop(0, stage, item, 0)
            return carry
        lax.fori_loop(0, stages, stage_body, 0)

        def padc(c, carry):
            cvec = jnp.full((16,), c, jnp.int32)
            p = jnp.minimum(jnp.min(plsc.load_gather(cnt, [cvec])), CAP)
            p2 = jnp.minimum(((p + SBE - 1) // SBE) * SBE, CAP)

            def padi(t, carry2):
                q = c * CAP + p + t * 16 + iota
                m = q < c * CAP + p2
                qh = lax.shift_right_logical(q, 7)
                ql = q & (SBE - 1)
                plsc.store_scatter(
                    bt, [qh, ql],
                    wid * 997 + c * 131 + t * 16 + iota, mask=m)
                plsc.store_scatter(
                    bj, [qh, ql],
                    jnp.full((16,), c * CH + CH + (wid & (GROWS - 1)),
                             jnp.int32), mask=m)
                return carry2
            lax.fori_loop(0, (SBE + 15) // 16, padi, 0)
            plsc.store_scatter(cnt, [jnp.full((16,), c, jnp.int32)],
                               jnp.full((16,), p2, jnp.int32),
                               mask=iota == 0)
            return carry
        lax.fori_loop(0, NCHUNK, padc, 0)

        def cw(i, carry):
            cout[0, pl.ds(i * 16, 16)] = cnt[pl.ds(i * 16, 16)]
            return carry
        lax.fori_loop(0, 128 // 16, cw, 0)

        pltpu.sync_copy(bt, bt_hbm.at[wid])
        pltpu.sync_copy(bj, bj_hbm.at[wid])
        pltpu.sync_copy(cout, cnt_hbm.at[wid])

    return bk(idx_ji)


def _sc_agg(v_att, att8, bins_tid, bins_ji, counts):
    stripe = CH // NS
    zeros = jnp.zeros((stripe, HID), jnp.float32)
    zeros8 = jnp.zeros((stripe, HEADS), jnp.float32)
    mesh = plsc.VectorSubcoreMesh(**_SC_MESH)

    @functools.partial(
        pl.kernel,
        out_type=[jax.ShapeDtypeStruct((E, HID), jnp.float32),
                  jax.ShapeDtypeStruct((E, HEADS), jnp.float32)],
        mesh=mesh,
        compiler_params=pltpu.CompilerParams(needs_layout_passes=False,
                                             use_tc_tiling_on_sc=False),
        scratch_types=[
            pltpu.VMEM_SHARED((CH + GROWS, HID), jnp.float32),
            pltpu.VMEM_SHARED((CH + GROWS, HEADS), jnp.float32),
            pltpu.VMEM((NW, 1, 128), jnp.int32),
            pltpu.VMEM((SBE,), jnp.int32),
            pltpu.VMEM((SBE,), jnp.int32),
            pltpu.VMEM((SBE,), jnp.int32),
            pltpu.VMEM((SBE, HID), jnp.float32),
            pltpu.VMEM((SBE, HEADS), jnp.float32),
            pltpu.VMEM((SBE,), jnp.int32),
            pltpu.VMEM((SBE,), jnp.int32),
            pltpu.VMEM((SBE,), jnp.int32),
            pltpu.VMEM((SBE, HID), jnp.float32),
            pltpu.VMEM((SBE, HEADS), jnp.float32),
            pltpu.SemaphoreType.DMA,
            pltpu.SemaphoreType.DMA,
            pltpu.SemaphoreType.DMA,
            pltpu.SemaphoreType.DMA,
        ],
    )
    def ek(vatt_hbm, att_hbm, bt_hbm, bj_hbm, cnt_hbm, z_hbm, z16_hbm,
           out_hbm, att_out_hbm,
           slab, aslab, cbuf, tid_v, ji_v, rel_v, rows_v, arows_v,
           tid2_v, ji2_v, rel2_v, rows2_v, arows2_v,
           sem, asem, sem2, asem2):
        cid = lax.axis_index("c")
        sid = lax.axis_index("s")
        pltpu.sync_copy(cnt_hbm, cbuf)

        def chunk_body(cc, carry):
            c = cc * NC + cid
            cbase = c * CH

            pltpu.sync_copy(z_hbm, slab.at[pl.ds(sid * stripe, stripe)])
            pltpu.sync_copy(z16_hbm, aslab.at[pl.ds(sid * stripe, stripe)])

            @pl.when(sid == 0)
            def _():
                pltpu.sync_copy(z_hbm.at[pl.ds(0, GROWS)],
                                slab.at[pl.ds(CH, GROWS)])
                pltpu.sync_copy(z16_hbm.at[pl.ds(0, GROWS)],
                                aslab.at[pl.ds(CH, GROWS)])
            plsc.subcore_barrier()

            w0 = sid * 2
            npad0 = jnp.min(plsc.load_gather(
                cbuf, [jnp.full((16,), w0, jnp.int32),
                       jnp.zeros((16,), jnp.int32),
                       jnp.full((16,), c, jnp.int32)]))
            npad1 = jnp.min(plsc.load_gather(
                cbuf, [jnp.full((16,), w0 + 1, jnp.int32),
                       jnp.zeros((16,), jnp.int32),
                       jnp.full((16,), c, jnp.int32)]))
            nb0 = lax.shift_right_logical(npad0, 7)
            nb = nb0 + lax.shift_right_logical(npad1, 7)

            def row_of(b):
                w = jnp.where(b < nb0, w0, w0 + 1)
                r = jnp.where(b < nb0, b, b - nb0) + c * CAPB
                return w, r

            def fire(b, tid_b, ji_b, rel_b, rows_b, arows_b, sem_b, asem_b):
                w, r = row_of(b)
                pltpu.sync_copy(bt_hbm.at[w].at[r], tid_b)
                pltpu.sync_copy(bj_hbm.at[w].at[r], ji_b)

                def torel(i, carry4):
                    rel_b[pl.ds(i * 16, 16)] = (
                        ji_b[pl.ds(i * 16, 16)] - cbase)
                    return carry4
                lax.fori_loop(0, SBE // 16, torel, 0)
                pltpu.async_copy(vatt_hbm.at[tid_b], rows_b, sem_b)
                pltpu.async_copy(att_hbm.at[tid_b], arows_b, asem_b)

            def drain_sc(tid_b, rel_b, rows_b, arows_b, sem_b, asem_b):
                pltpu.make_async_copy(vatt_hbm.at[tid_b], rows_b,
                                      sem_b).wait()
                pltpu.make_async_copy(att_hbm.at[tid_b], arows_b,
                                      asem_b).wait()
                pltpu.sync_copy(rows_b, slab.at[rel_b], add=True)
                pltpu.sync_copy(arows_b, aslab.at[rel_b], add=True)

            @pl.when(nb > 0)
            def _():
                fire(0, tid_v, ji_v, rel_v, rows_v, arows_v, sem, asem)

            def batch2(j, carry3):
                b0 = j * 2
                b1 = b0 + 1
                fire(b1, tid2_v, ji2_v, rel2_v, rows2_v, arows2_v,
                     sem2, asem2)
                drain_sc(tid_v, rel_v, rows_v, arows_v, sem, asem)

                @pl.when(b0 + 2 < nb)
                def _():
                    fire(b0 + 2, tid_v, ji_v, rel_v, rows_v, arows_v,
                         sem, asem)
                drain_sc(tid2_v, rel2_v, rows2_v, arows2_v, sem2, asem2)
                return carry3
            lax.fori_loop(0, lax.shift_right_logical(nb, 1), batch2, 0)

            @pl.when(nb & 1 == 1)
            def _():
                drain_sc(tid_v, rel_v, rows_v, arows_v, sem, asem)
            plsc.subcore_barrier()

            rbase = cbase + sid * stripe

            @pl.when(rbase < E)
            def _():
                pltpu.sync_copy(slab.at[pl.ds(sid * stripe, stripe)],
                                out_hbm.at[pl.ds(rbase, stripe)])
                pltpu.sync_copy(aslab.at[pl.ds(sid * stripe, stripe)],
                                att_out_hbm.at[pl.ds(rbase, stripe)])
            return carry

        lax.fori_loop(0, NCHUNK // NC, chunk_body, 0)

    return ek(v_att, att8, bins_tid, bins_ji, counts, zeros, zeros8)


def _pad_rows(x, mult=8):
    pad = (-x.shape[0]) % mult
    return jnp.pad(x, ((0, pad), (0, 0))) if pad else x


def kernel(atom_feature, edge_feature, src, dst, idx_kj, idx_ji, W_i,
           Wv0, Wk0, Wq0, r1w0, r1b0, r2w0, r2b0,
           Wv1, Wk1, Wq1, r1w1, r1b1, r2w1, r2b1,
           W_o, b_o):
    AF = atom_feature.shape[1]

    anode = _mm(atom_feature, W_i[:AF])
    feats = _mm(jnp.pad(edge_feature, ((0, 0), (0, 2))),
                _pad_rows(W_i[AF:]))
    feats = _relu(_sc_gather(anode, src) + feats)
    bins_tid, bins_ji, counts = _sc_bin(idx_ji)

    layers = [(Wv0, Wk0, Wq0, r1w0, r1b0, r2w0, r2b0),
              (Wv1, Wk1, Wq1, r1w1, r1b1, r2w1, r2b1)]
    for (Wv, Wk, Wq, r1w, r1b, r2w, r2b) in layers:
        q, k, v = _mm3(feats, Wq, Wk, Wv)
        qg = _sc_gather(q, idx_kj)
        kg = _sc_gather(k, idx_ji)
        vg = _sc_gather(v, idx_kj)
        att, v_att = _att_vatt(qg, kg, vg)
        agg, att_all = _sc_agg(v_att, att, bins_tid, bins_ji, counts)
        feats = _residual(agg, att_all, v, r1w, r1b, r2w, r2b)

    fparts = _sc_scatter_rows(feats, dst, N)
    out = _out_proj(jnp.pad(atom_feature, ((0, 0), (0, 3))),
                    fparts[0][:N], fparts[1][:N],
                    jnp.pad(W_o[:AF], ((0, 3), (0, 0))), W_o[AF:], b_o)
    return out
